# Initial kernel scaffold; baseline (speedup 1.0000x reference)
import functools

import jax
import jax.numpy as jnp
from jax import lax
from jax.experimental import pallas as pl
from jax.experimental.pallas import tpu as pltpu
from jax.experimental.pallas import tpu_sc as plsc

BLK = 1000


def _mm_kernel(x_ref, w_ref, o0, o1, o2):
    y = jnp.dot(x_ref[...], w_ref[...], preferred_element_type=jnp.float32)
    o0[...] = y[:, 0:32]
    o1[...] = y[:, 32:64]
    o2[...] = y[:, 64:96]


def _premul3(x, w_cat):
    n, d = x.shape
    grid = n // BLK
    outs = [jax.ShapeDtypeStruct((n, 32), jnp.float32)] * 3
    return pl.pallas_call(
        _mm_kernel,
        grid=(grid,),
        in_specs=[
            pl.BlockSpec((BLK, d), lambda i: (i, 0)),
            pl.BlockSpec((d, 96), lambda i: (0, 0)),
        ],
        out_specs=[pl.BlockSpec((BLK, 32), lambda i: (i, 0))] * 3,
        out_shape=outs,
    )(x, w_cat)


def _segsum3(y0, y1, y2, s0, d0, s1, d1, s2, d2):
    n = y0.shape[0]
    e = s0.shape[0]
    info = plsc.get_sparse_core_info()
    nc, ns = info.num_cores, info.num_subcores
    nw = nc * ns
    epw = e // nw
    ch = 80
    nch = epw // ch
    assert ch * nch == epw and epw * nw == e
    rpt = n // ns

    @functools.partial(
        pl.kernel,
        out_type=[jax.ShapeDtypeStruct((nc, n, 32), jnp.float32)] * 3,
        mesh=plsc.VectorSubcoreMesh(core_axis_name="c", subcore_axis_name="s"),
        scratch_types=[
            pltpu.VMEM_SHARED((n, 32), jnp.float32),
            pltpu.VMEM_SHARED((n, 32), jnp.float32),
            pltpu.VMEM_SHARED((n, 32), jnp.float32),
            pltpu.VMEM((ch,), jnp.int32),
            pltpu.VMEM((ch,), jnp.int32),
            pltpu.VMEM((ch, 32), jnp.float32),
            pltpu.VMEM((rpt, 32), jnp.float32),
            pltpu.SemaphoreType.DMA,
        ],
    )
    def k(y0h, y1h, y2h, s0h, d0h, s1h, d1h, s2h, d2h,
          o0, o1, o2, a0, a1, a2, sidx, didx, rows, zbuf, sem):
        cid = lax.axis_index("c")
        sid = lax.axis_index("s")
        wid = sid * nc + cid

        zero16 = jnp.zeros((16,), jnp.float32)

        def zb(i, carry):
            zbuf[i, pl.ds(0, 16)] = zero16
            zbuf[i, pl.ds(16, 16)] = zero16
            return carry

        lax.fori_loop(0, rpt, zb, 0)
        r0 = sid * rpt
        for a in (a0, a1, a2):
            pltpu.sync_copy(zbuf, a.at[pl.ds(r0, rpt)])
        plsc.subcore_barrier()

        base = wid * epw
        for yh, sh, dh, a in ((y0h, s0h, d0h, a0),
                              (y1h, s1h, d1h, a1),
                              (y2h, s2h, d2h, a2)):
            def body(c, carry):
                off = base + c * ch
                pltpu.sync_copy(sh.at[pl.ds(off, ch)], sidx)
                pltpu.sync_copy(dh.at[pl.ds(off, ch)], didx)
                pltpu.async_copy(yh.at[sidx], rows, sem).wait()
                pltpu.sync_copy(rows, a.at[didx], add=True)
                return carry

            lax.fori_loop(0, nch, body, 0)
        plsc.subcore_barrier()

        for a, o in ((a0, o0), (a1, o1), (a2, o2)):
            pltpu.sync_copy(a.at[pl.ds(r0, rpt)], o.at[cid, pl.ds(r0, rpt)])

    return k(y0, y1, y2, s0, d0, s1, d1, s2, d2)


def _mid_kernel(y0, y1, y2, p0, p1, p2, b1s, w2s, b2s,
                m1w1, m1b1, m1w2, m1b2, w1cat2, o0, o1, o2):
    ts = []
    for i, (y, p) in enumerate(((y0, p0), (y1, p1), (y2, p2))):
        pre = y[...] + p[0] + p[1] + b1s[pl.ds(i, 1)]
        t = jnp.dot(jnp.maximum(pre, 0.0), w2s[i],
                    preferred_element_type=jnp.float32) + b2s[pl.ds(i, 1)]
        ts.append(t)
    tcat = jnp.concatenate(ts, axis=1)
    u = jnp.maximum(jnp.dot(tcat, m1w1[...],
                            preferred_element_type=jnp.float32) + m1b1[...], 0.0)
    u = jnp.dot(u, m1w2[...], preferred_element_type=jnp.float32) + m1b2[...]
    z = jnp.dot(u, w1cat2[...], preferred_element_type=jnp.float32)
    o0[...] = z[:, 0:32]
    o1[...] = z[:, 32:64]
    o2[...] = z[:, 64:96]


def _mid(y0, y1, y2, p0, p1, p2, b1s, w2s, b2s, m1w1, m1b1, m1w2, m1b2, w1cat2):
    n = y0.shape[0]
    grid = n // BLK
    yspec = pl.BlockSpec((BLK, 32), lambda i: (i, 0))
    pspec = pl.BlockSpec((2, BLK, 32), lambda i: (0, i, 0))
    full = lambda s: pl.BlockSpec(s, lambda i: tuple(0 for _ in s))
    return pl.pallas_call(
        _mid_kernel,
        grid=(grid,),
        in_specs=[yspec] * 3 + [pspec] * 3 + [
            full((3, 32)), full((3, 32, 32)), full((3, 32)),
            full((96, 32)), full((1, 32)), full((32, 32)), full((1, 32)),
            full((32, 96)),
        ],
        out_specs=[yspec] * 3,
        out_shape=[jax.ShapeDtypeStruct((n, 32), jnp.float32)] * 3,
    )(y0, y1, y2, p0, p1, p2, b1s, w2s, b2s, m1w1, m1b1, m1w2, m1b2, w1cat2)


def _fin_kernel(z0, z1, z2, q0, q1, q2, batch_ref, b1s, w2s, b2s,
                m2w1, m2b1, m2w2, m2b2, linw, linb, out_ref, acc):
    i = pl.program_id(0)
    nblk = pl.num_programs(0)
    xs = []
    for k, (z, q) in enumerate(((z0, q0), (z1, q1), (z2, q2))):
        pre = z[...] + q[0] + q[1] + b1s[pl.ds(k, 1)]
        t = jnp.dot(jnp.maximum(pre, 0.0), w2s[k],
                    preferred_element_type=jnp.float32) + b2s[pl.ds(k, 1)]
        xs.append(jnp.maximum(t, 0.0))
    cat = jnp.concatenate(xs, axis=1)
    v = jnp.maximum(jnp.dot(cat, m2w1[...],
                            preferred_element_type=jnp.float32) + m2b1[...], 0.0)
    v = jnp.dot(v, m2w2[...], preferred_element_type=jnp.float32) + m2b2[...]
    bb = batch_ref[0]
    g = acc.shape[0]
    oh_t = jnp.where(
        jax.lax.broadcasted_iota(jnp.int32, (g, v.shape[0]), 0) == bb,
        1.0, 0.0)
    part = jnp.dot(oh_t, v, preferred_element_type=jnp.float32)

    @pl.when(i == 0)
    def _():
        acc[...] = jnp.zeros_like(acc)

    acc[...] += part

    @pl.when(i == nblk - 1)
    def _():
        out_ref[...] = jnp.dot(acc[...], linw[...],
                               preferred_element_type=jnp.float32) + linb[...]


def _final(z0, z1, z2, q0, q1, q2, batch3d, b1s, w2s, b2s,
           m2w1, m2b1, m2w2, m2b2, linw, linb, g):
    n = z0.shape[0]
    grid = n // BLK
    zspec = pl.BlockSpec((BLK, 32), lambda i: (i, 0))
    pspec = pl.BlockSpec((2, BLK, 32), lambda i: (0, i, 0))
    full = lambda s: pl.BlockSpec(s, lambda i: tuple(0 for _ in s))
    return pl.pallas_call(
        _fin_kernel,
        grid=(grid,),
        in_specs=[zspec] * 3 + [pspec] * 3 + [
            pl.BlockSpec((1, 1, BLK), lambda i: (i, 0, 0)),
            full((3, 32)), full((3, 32, 32)), full((3, 32)),
            full((96, 32)), full((1, 32)), full((32, 32)), full((1, 32)),
            full((32, 1)), full((1, 1)),
        ],
        out_specs=full((g, 1)),
        out_shape=jax.ShapeDtypeStruct((g, 1), jnp.float32),
        scratch_shapes=[pltpu.VMEM((g, 32), jnp.float32)],
    )(z0, z1, z2, q0, q1, q2, batch3d, b1s, w2s, b2s,
      m2w1, m2b1, m2w2, m2b2, linw, linb)


def kernel(x, edge_index_0, edge_index_1, edge_index_2, batch,
           c11_W1, c11_b1, c11_W2, c11_b2,
           c12_W1, c12_b1, c12_W2, c12_b2,
           c13_W1, c13_b1, c13_W2, c13_b2,
           c21_W1, c21_b1, c21_W2, c21_b2,
           c22_W1, c22_b1, c22_W2, c22_b2,
           c23_W1, c23_b1, c23_W2, c23_b2,
           mlp1_W1, mlp1_b1, mlp1_W2, mlp1_b2,
           mlp2_W1, mlp2_b1, mlp2_W2, mlp2_b2,
           lin_W, lin_b):
    n = x.shape[0]
    g = 64

    s0, d0 = edge_index_0[0], edge_index_0[1]
    s1, d1 = edge_index_1[0], edge_index_1[1]
    s2, d2 = edge_index_2[0], edge_index_2[1]

    w1cat = jnp.concatenate([c11_W1, c12_W1, c13_W1], axis=1)
    w1cat2 = jnp.concatenate([c21_W1, c22_W1, c23_W1], axis=1)
    b1s_1 = jnp.stack([c11_b1, c12_b1, c13_b1])
    w2s_1 = jnp.stack([c11_W2, c12_W2, c13_W2])
    b2s_1 = jnp.stack([c11_b2, c12_b2, c13_b2])
    b1s_2 = jnp.stack([c21_b1, c22_b1, c23_b1])
    w2s_2 = jnp.stack([c21_W2, c22_W2, c23_W2])
    b2s_2 = jnp.stack([c21_b2, c22_b2, c23_b2])

    y0, y1, y2 = _premul3(x, w1cat)
    p0, p1, p2 = _segsum3(y0, y1, y2, s0, d0, s1, d1, s2, d2)
    z0, z1, z2 = _mid(y0, y1, y2, p0, p1, p2, b1s_1, w2s_1, b2s_1,
                      mlp1_W1, mlp1_b1.reshape(1, 32), mlp1_W2,
                      mlp1_b2.reshape(1, 32), w1cat2)

    q0, q1, q2 = _segsum3(z0, z1, z2, s0, d0, s1, d1, s2, d2)
    batch3d = batch.reshape(n // BLK, 1, BLK)
    out = _final(z0, z1, z2, q0, q1, q2, batch3d, b1s_2, w2s_2, b2s_2,
                 mlp2_W1, mlp2_b1.reshape(1, 32), mlp2_W2,
                 mlp2_b2.reshape(1, 32), lin_W, lin_b.reshape(1, 1), g)
    return jnp.squeeze(out, axis=-1)


# speedup vs baseline: 4.2696x; 4.2696x over previous
import functools

import jax
import jax.numpy as jnp
from jax import lax
from jax.experimental import pallas as pl
from jax.experimental.pallas import tpu as pltpu
from jax.experimental.pallas import tpu_sc as plsc

BLK = 1000


def _mm_kernel(x_ref, w_ref, o0, o1, o2):
    y = jnp.dot(x_ref[...], w_ref[...], preferred_element_type=jnp.float32)
    o0[...] = y[:, 0:32]
    o1[...] = y[:, 32:64]
    o2[...] = y[:, 64:96]


def _premul3(x, w_cat):
    n, d = x.shape
    grid = n // BLK
    outs = [jax.ShapeDtypeStruct((n, 32), jnp.float32)] * 3
    return pl.pallas_call(
        _mm_kernel,
        grid=(grid,),
        in_specs=[
            pl.BlockSpec((BLK, d), lambda i: (i, 0)),
            pl.BlockSpec((d, 96), lambda i: (0, 0)),
        ],
        out_specs=[pl.BlockSpec((BLK, 32), lambda i: (i, 0))] * 3,
        out_shape=outs,
    )(x, w_cat)


def _segsum3(y0, y1, y2, s0, d0, s1, d1, s2, d2):
    n = y0.shape[0]
    e = s0.shape[0]
    info = plsc.get_sparse_core_info()
    nc, ns = info.num_cores, info.num_subcores
    nw = nc * ns
    epw = e // nw
    ch = 80
    nch = epw // ch
    assert ch * nch == epw and epw * nw == e
    rpt = (-(-n // ns) + 7) // 8 * 8
    n_pad = rpt * ns

    @functools.partial(
        pl.kernel,
        out_type=[jax.ShapeDtypeStruct((nc, n_pad, 32), jnp.float32)] * 3,
        mesh=plsc.VectorSubcoreMesh(core_axis_name="c", subcore_axis_name="s"),
        scratch_types=[
            pltpu.VMEM_SHARED((n_pad, 32), jnp.float32),
            pltpu.VMEM_SHARED((n_pad, 32), jnp.float32),
            pltpu.VMEM_SHARED((n_pad, 32), jnp.float32),
            pltpu.VMEM((ch,), jnp.int32),
            pltpu.VMEM((ch,), jnp.int32),
            pltpu.VMEM((ch, 32), jnp.float32),
            pltpu.VMEM((rpt, 32), jnp.float32),
            pltpu.SemaphoreType.DMA,
        ],
        compiler_params=pltpu.CompilerParams(use_tc_tiling_on_sc=False),
    )
    def k(y0h, y1h, y2h, s0h, d0h, s1h, d1h, s2h, d2h,
          o0, o1, o2, a0, a1, a2, sidx, didx, rows, zbuf, sem):
        cid = lax.axis_index("c")
        sid = lax.axis_index("s")
        wid = sid * nc + cid

        zero16 = jnp.zeros((16,), jnp.float32)

        def zb(i, carry):
            zbuf[i, pl.ds(0, 16)] = zero16
            zbuf[i, pl.ds(16, 16)] = zero16
            return carry

        lax.fori_loop(0, rpt, zb, 0)
        r0 = sid * rpt
        for a in (a0, a1, a2):
            pltpu.sync_copy(zbuf, a.at[pl.ds(r0, rpt)])
        plsc.subcore_barrier()

        base = wid * epw
        for yh, sh, dh, a in ((y0h, s0h, d0h, a0),
                              (y1h, s1h, d1h, a1),
                              (y2h, s2h, d2h, a2)):
            def body(c, carry):
                off = base + c * ch
                pltpu.sync_copy(sh.at[pl.ds(off, ch)], sidx)
                pltpu.sync_copy(dh.at[pl.ds(off, ch)], didx)
                pltpu.async_copy(yh.at[sidx], rows, sem).wait()
                pltpu.sync_copy(rows, a.at[didx], add=True)
                return carry

            lax.fori_loop(0, nch, body, 0)
        plsc.subcore_barrier()

        for a, o in ((a0, o0), (a1, o1), (a2, o2)):
            pltpu.sync_copy(a.at[pl.ds(r0, rpt)], o.at[cid, pl.ds(r0, rpt)])

    return k(y0, y1, y2, s0, d0, s1, d1, s2, d2)


def _mid_kernel(y0, y1, y2, p0, p1, p2, b1s, w2s, b2s,
                m1w1, m1b1, m1w2, m1b2, w1cat2, o0, o1, o2):
    ts = []
    for i, (y, p) in enumerate(((y0, p0), (y1, p1), (y2, p2))):
        pre = y[...] + p[0] + p[1] + b1s[pl.ds(i, 1)]
        t = jnp.dot(jnp.maximum(pre, 0.0), w2s[i],
                    preferred_element_type=jnp.float32) + b2s[pl.ds(i, 1)]
        ts.append(t)
    tcat = jnp.concatenate(ts, axis=1)
    u = jnp.maximum(jnp.dot(tcat, m1w1[...],
                            preferred_element_type=jnp.float32) + m1b1[...], 0.0)
    u = jnp.dot(u, m1w2[...], preferred_element_type=jnp.float32) + m1b2[...]
    z = jnp.dot(u, w1cat2[...], preferred_element_type=jnp.float32)
    o0[...] = z[:, 0:32]
    o1[...] = z[:, 32:64]
    o2[...] = z[:, 64:96]


def _mid(y0, y1, y2, p0, p1, p2, b1s, w2s, b2s, m1w1, m1b1, m1w2, m1b2, w1cat2):
    n = y0.shape[0]
    grid = n // BLK
    yspec = pl.BlockSpec((BLK, 32), lambda i: (i, 0))
    pspec = pl.BlockSpec((2, BLK, 32), lambda i: (0, i, 0))
    full = lambda s: pl.BlockSpec(s, lambda i: tuple(0 for _ in s))
    return pl.pallas_call(
        _mid_kernel,
        grid=(grid,),
        in_specs=[yspec] * 3 + [pspec] * 3 + [
            full((3, 32)), full((3, 32, 32)), full((3, 32)),
            full((96, 32)), full((1, 32)), full((32, 32)), full((1, 32)),
            full((32, 96)),
        ],
        out_specs=[yspec] * 3,
        out_shape=[jax.ShapeDtypeStruct((n, 32), jnp.float32)] * 3,
    )(y0, y1, y2, p0, p1, p2, b1s, w2s, b2s, m1w1, m1b1, m1w2, m1b2, w1cat2)


def _fin_kernel(z0, z1, z2, q0, q1, q2, batch_ref, b1s, w2s, b2s,
                m2w1, m2b1, m2w2, m2b2, linw, linb, out_ref, acc):
    i = pl.program_id(0)
    nblk = pl.num_programs(0)
    xs = []
    for k, (z, q) in enumerate(((z0, q0), (z1, q1), (z2, q2))):
        pre = z[...] + q[0] + q[1] + b1s[pl.ds(k, 1)]
        t = jnp.dot(jnp.maximum(pre, 0.0), w2s[k],
                    preferred_element_type=jnp.float32) + b2s[pl.ds(k, 1)]
        xs.append(jnp.maximum(t, 0.0))
    cat = jnp.concatenate(xs, axis=1)
    v = jnp.maximum(jnp.dot(cat, m2w1[...],
                            preferred_element_type=jnp.float32) + m2b1[...], 0.0)
    v = jnp.dot(v, m2w2[...], preferred_element_type=jnp.float32) + m2b2[...]
    bb = batch_ref[0]
    g = acc.shape[0]
    oh_t = jnp.where(
        jax.lax.broadcasted_iota(jnp.int32, (g, v.shape[0]), 0) == bb,
        1.0, 0.0)
    part = jnp.dot(oh_t, v, preferred_element_type=jnp.float32)

    @pl.when(i == 0)
    def _():
        acc[...] = jnp.zeros_like(acc)

    acc[...] += part

    @pl.when(i == nblk - 1)
    def _():
        out_ref[...] = jnp.dot(acc[...], linw[...],
                               preferred_element_type=jnp.float32) + linb[...]


def _final(z0, z1, z2, q0, q1, q2, batch3d, b1s, w2s, b2s,
           m2w1, m2b1, m2w2, m2b2, linw, linb, g):
    n = z0.shape[0]
    grid = n // BLK
    zspec = pl.BlockSpec((BLK, 32), lambda i: (i, 0))
    pspec = pl.BlockSpec((2, BLK, 32), lambda i: (0, i, 0))
    full = lambda s: pl.BlockSpec(s, lambda i: tuple(0 for _ in s))
    return pl.pallas_call(
        _fin_kernel,
        grid=(grid,),
        in_specs=[zspec] * 3 + [pspec] * 3 + [
            pl.BlockSpec((1, 1, BLK), lambda i: (i, 0, 0)),
            full((3, 32)), full((3, 32, 32)), full((3, 32)),
            full((96, 32)), full((1, 32)), full((32, 32)), full((1, 32)),
            full((32, 1)), full((1, 1)),
        ],
        out_specs=full((g, 1)),
        out_shape=jax.ShapeDtypeStruct((g, 1), jnp.float32),
        scratch_shapes=[pltpu.VMEM((g, 32), jnp.float32)],
    )(z0, z1, z2, q0, q1, q2, batch3d, b1s, w2s, b2s,
      m2w1, m2b1, m2w2, m2b2, linw, linb)


def kernel(x, edge_index_0, edge_index_1, edge_index_2, batch,
           c11_W1, c11_b1, c11_W2, c11_b2,
           c12_W1, c12_b1, c12_W2, c12_b2,
           c13_W1, c13_b1, c13_W2, c13_b2,
           c21_W1, c21_b1, c21_W2, c21_b2,
           c22_W1, c22_b1, c22_W2, c22_b2,
           c23_W1, c23_b1, c23_W2, c23_b2,
           mlp1_W1, mlp1_b1, mlp1_W2, mlp1_b2,
           mlp2_W1, mlp2_b1, mlp2_W2, mlp2_b2,
           lin_W, lin_b):
    n = x.shape[0]
    g = 64

    s0, d0 = edge_index_0[0], edge_index_0[1]
    s1, d1 = edge_index_1[0], edge_index_1[1]
    s2, d2 = edge_index_2[0], edge_index_2[1]

    w1cat = jnp.concatenate([c11_W1, c12_W1, c13_W1], axis=1)
    w1cat2 = jnp.concatenate([c21_W1, c22_W1, c23_W1], axis=1)
    b1s_1 = jnp.stack([c11_b1, c12_b1, c13_b1])
    w2s_1 = jnp.stack([c11_W2, c12_W2, c13_W2])
    b2s_1 = jnp.stack([c11_b2, c12_b2, c13_b2])
    b1s_2 = jnp.stack([c21_b1, c22_b1, c23_b1])
    w2s_2 = jnp.stack([c21_W2, c22_W2, c23_W2])
    b2s_2 = jnp.stack([c21_b2, c22_b2, c23_b2])

    y0, y1, y2 = _premul3(x, w1cat)
    p0, p1, p2 = _segsum3(y0, y1, y2, s0, d0, s1, d1, s2, d2)
    z0, z1, z2 = _mid(y0, y1, y2, p0, p1, p2, b1s_1, w2s_1, b2s_1,
                      mlp1_W1, mlp1_b1.reshape(1, 32), mlp1_W2,
                      mlp1_b2.reshape(1, 32), w1cat2)

    q0, q1, q2 = _segsum3(z0, z1, z2, s0, d0, s1, d1, s2, d2)
    batch3d = batch.reshape(n // BLK, 1, BLK)
    out = _final(z0, z1, z2, q0, q1, q2, batch3d, b1s_2, w2s_2, b2s_2,
                 mlp2_W1, mlp2_b1.reshape(1, 32), mlp2_W2,
                 mlp2_b2.reshape(1, 32), lin_W, lin_b.reshape(1, 1), g)
    return jnp.squeeze(out, axis=-1)


# speedup vs baseline: 9.3540x; 2.1908x over previous
import functools

import jax
import jax.numpy as jnp
from jax import lax
from jax.experimental import pallas as pl
from jax.experimental.pallas import tpu as pltpu
from jax.experimental.pallas import tpu_sc as plsc

BLK = 1000


def _mm_kernel(x_ref, w_ref, o0, o1, o2):
    y = jnp.dot(x_ref[...], w_ref[...], preferred_element_type=jnp.float32)
    o0[...] = y[:, 0:32]
    o1[...] = y[:, 32:64]
    o2[...] = y[:, 64:96]


def _premul3(x, w_cat):
    n, d = x.shape
    grid = n // BLK
    outs = [jax.ShapeDtypeStruct((n, 32), jnp.float32)] * 3
    return pl.pallas_call(
        _mm_kernel,
        grid=(grid,),
        in_specs=[
            pl.BlockSpec((BLK, d), lambda i: (i, 0)),
            pl.BlockSpec((d, 96), lambda i: (0, 0)),
        ],
        out_specs=[pl.BlockSpec((BLK, 32), lambda i: (i, 0))] * 3,
        out_shape=outs,
    )(x, w_cat)


def _segsum3(y0, y1, y2, s0, d0, s1, d1, s2, d2):
    n = y0.shape[0]
    e = s0.shape[0]
    info = plsc.get_sparse_core_info()
    nc, ns = info.num_cores, info.num_subcores
    nw = nc * ns
    epw = e // nw
    ch = 100
    nch = epw // ch
    assert ch * nch == epw and epw * nw == e and nch % 2 == 0
    rpt = (-(-n // ns) + 7) // 8 * 8
    n_pad = rpt * ns

    s0r, d0r = s0.reshape(nw, nch, ch), d0.reshape(nw, nch, ch)
    s1r, d1r = s1.reshape(nw, nch, ch), d1.reshape(nw, nch, ch)
    s2r, d2r = s2.reshape(nw, nch, ch), d2.reshape(nw, nch, ch)

    @functools.partial(
        pl.kernel,
        out_type=[jax.ShapeDtypeStruct((nc, n_pad, 32), jnp.float32)] * 3,
        mesh=plsc.VectorSubcoreMesh(core_axis_name="c", subcore_axis_name="s"),
        scratch_types=[
            pltpu.VMEM_SHARED((n_pad, 32), jnp.float32),
            pltpu.VMEM_SHARED((n_pad, 32), jnp.float32),
            pltpu.VMEM_SHARED((n_pad, 32), jnp.float32),
            [pltpu.VMEM((nch, ch), jnp.int32) for _ in range(6)],
            pltpu.VMEM((ch, 32), jnp.float32),
            pltpu.VMEM((ch, 32), jnp.float32),
            pltpu.SemaphoreType.DMA,
            pltpu.SemaphoreType.DMA,
        ],
        compiler_params=pltpu.CompilerParams(use_tc_tiling_on_sc=False),
    )
    def k(y0h, y1h, y2h, s0h, d0h, s1h, d1h, s2h, d2h,
          o0, o1, o2, a0, a1, a2, idxs, rows_a, rows_b, gsa, gsb):
        si0, di0, si1, di1, si2, di2 = idxs
        cid = lax.axis_index("c")
        sid = lax.axis_index("s")
        wid = sid * nc + cid

        for hb, vm in ((s0h, si0), (d0h, di0), (s1h, si1),
                       (d1h, di1), (s2h, si2), (d2h, di2)):
            pltpu.sync_copy(hb.at[wid], vm)

        zero16 = jnp.zeros((16,), jnp.float32)

        def zb(i, carry):
            rows_a[i, pl.ds(0, 16)] = zero16
            rows_a[i, pl.ds(16, 16)] = zero16
            return carry

        lax.fori_loop(0, ch, zb, 0)
        r0 = sid * rpt
        nfull, rem = divmod(rpt, ch)
        for a in (a0, a1, a2):
            for j in range(nfull):
                pltpu.sync_copy(rows_a, a.at[pl.ds(r0 + j * ch, ch)])
            if rem:
                pltpu.sync_copy(rows_a.at[pl.ds(0, rem)],
                                a.at[pl.ds(r0 + nfull * ch, rem)])
        plsc.subcore_barrier()

        for yh, si, di, a in ((y0h, si0, di0, a0),
                              (y1h, si1, di1, a1),
                              (y2h, si2, di2, a2)):
            pltpu.async_copy(yh.at[si.at[0]], rows_a, gsa)

            def pair(p, carry):
                c0 = 2 * p
                pltpu.make_async_copy(yh.at[si.at[c0]], rows_a, gsa).wait()
                pltpu.async_copy(yh.at[si.at[c0 + 1]], rows_b, gsb)
                pltpu.sync_copy(rows_a, a.at[di.at[c0]], add=True)
                pltpu.make_async_copy(yh.at[si.at[c0 + 1]], rows_b, gsb).wait()

                @pl.when(c0 + 2 < nch)
                def _():
                    pltpu.async_copy(yh.at[si.at[c0 + 2]], rows_a, gsa)

                pltpu.sync_copy(rows_b, a.at[di.at[c0 + 1]], add=True)
                return carry

            lax.fori_loop(0, nch // 2, pair, 0)
        plsc.subcore_barrier()

        for a, o in ((a0, o0), (a1, o1), (a2, o2)):
            pltpu.sync_copy(a.at[pl.ds(r0, rpt)], o.at[cid, pl.ds(r0, rpt)])

    return k(y0, y1, y2, s0r, d0r, s1r, d1r, s2r, d2r)


def _mid_kernel(y0, y1, y2, p0, p1, p2, b1s, w2s, b2s,
                m1w1, m1b1, m1w2, m1b2, w1cat2, o0, o1, o2):
    ts = []
    for i, (y, p) in enumerate(((y0, p0), (y1, p1), (y2, p2))):
        pre = y[...] + p[0] + p[1] + b1s[pl.ds(i, 1)]
        t = jnp.dot(jnp.maximum(pre, 0.0), w2s[i],
                    preferred_element_type=jnp.float32) + b2s[pl.ds(i, 1)]
        ts.append(t)
    tcat = jnp.concatenate(ts, axis=1)
    u = jnp.maximum(jnp.dot(tcat, m1w1[...],
                            preferred_element_type=jnp.float32) + m1b1[...], 0.0)
    u = jnp.dot(u, m1w2[...], preferred_element_type=jnp.float32) + m1b2[...]
    z = jnp.dot(u, w1cat2[...], preferred_element_type=jnp.float32)
    o0[...] = z[:, 0:32]
    o1[...] = z[:, 32:64]
    o2[...] = z[:, 64:96]


def _mid(y0, y1, y2, p0, p1, p2, b1s, w2s, b2s, m1w1, m1b1, m1w2, m1b2, w1cat2):
    n = y0.shape[0]
    grid = n // BLK
    yspec = pl.BlockSpec((BLK, 32), lambda i: (i, 0))
    pspec = pl.BlockSpec((2, BLK, 32), lambda i: (0, i, 0))
    full = lambda s: pl.BlockSpec(s, lambda i: tuple(0 for _ in s))
    return pl.pallas_call(
        _mid_kernel,
        grid=(grid,),
        in_specs=[yspec] * 3 + [pspec] * 3 + [
            full((3, 32)), full((3, 32, 32)), full((3, 32)),
            full((96, 32)), full((1, 32)), full((32, 32)), full((1, 32)),
            full((32, 96)),
        ],
        out_specs=[yspec] * 3,
        out_shape=[jax.ShapeDtypeStruct((n, 32), jnp.float32)] * 3,
    )(y0, y1, y2, p0, p1, p2, b1s, w2s, b2s, m1w1, m1b1, m1w2, m1b2, w1cat2)


def _fin_kernel(z0, z1, z2, q0, q1, q2, batch_ref, b1s, w2s, b2s,
                m2w1, m2b1, m2w2, m2b2, linw, linb, out_ref, acc):
    i = pl.program_id(0)
    nblk = pl.num_programs(0)
    xs = []
    for k, (z, q) in enumerate(((z0, q0), (z1, q1), (z2, q2))):
        pre = z[...] + q[0] + q[1] + b1s[pl.ds(k, 1)]
        t = jnp.dot(jnp.maximum(pre, 0.0), w2s[k],
                    preferred_element_type=jnp.float32) + b2s[pl.ds(k, 1)]
        xs.append(jnp.maximum(t, 0.0))
    cat = jnp.concatenate(xs, axis=1)
    v = jnp.maximum(jnp.dot(cat, m2w1[...],
                            preferred_element_type=jnp.float32) + m2b1[...], 0.0)
    v = jnp.dot(v, m2w2[...], preferred_element_type=jnp.float32) + m2b2[...]
    bb = batch_ref[0]
    g = acc.shape[0]
    oh_t = jnp.where(
        jax.lax.broadcasted_iota(jnp.int32, (g, v.shape[0]), 0) == bb,
        1.0, 0.0)
    part = jnp.dot(oh_t, v, preferred_element_type=jnp.float32)

    @pl.when(i == 0)
    def _():
        acc[...] = jnp.zeros_like(acc)

    acc[...] += part

    @pl.when(i == nblk - 1)
    def _():
        out_ref[...] = jnp.dot(acc[...], linw[...],
                               preferred_element_type=jnp.float32) + linb[...]


def _final(z0, z1, z2, q0, q1, q2, batch3d, b1s, w2s, b2s,
           m2w1, m2b1, m2w2, m2b2, linw, linb, g):
    n = z0.shape[0]
    grid = n // BLK
    zspec = pl.BlockSpec((BLK, 32), lambda i: (i, 0))
    pspec = pl.BlockSpec((2, BLK, 32), lambda i: (0, i, 0))
    full = lambda s: pl.BlockSpec(s, lambda i: tuple(0 for _ in s))
    return pl.pallas_call(
        _fin_kernel,
        grid=(grid,),
        in_specs=[zspec] * 3 + [pspec] * 3 + [
            pl.BlockSpec((1, 1, BLK), lambda i: (i, 0, 0)),
            full((3, 32)), full((3, 32, 32)), full((3, 32)),
            full((96, 32)), full((1, 32)), full((32, 32)), full((1, 32)),
            full((32, 1)), full((1, 1)),
        ],
        out_specs=full((g, 1)),
        out_shape=jax.ShapeDtypeStruct((g, 1), jnp.float32),
        scratch_shapes=[pltpu.VMEM((g, 32), jnp.float32)],
    )(z0, z1, z2, q0, q1, q2, batch3d, b1s, w2s, b2s,
      m2w1, m2b1, m2w2, m2b2, linw, linb)


def kernel(x, edge_index_0, edge_index_1, edge_index_2, batch,
           c11_W1, c11_b1, c11_W2, c11_b2,
           c12_W1, c12_b1, c12_W2, c12_b2,
           c13_W1, c13_b1, c13_W2, c13_b2,
           c21_W1, c21_b1, c21_W2, c21_b2,
           c22_W1, c22_b1, c22_W2, c22_b2,
           c23_W1, c23_b1, c23_W2, c23_b2,
           mlp1_W1, mlp1_b1, mlp1_W2, mlp1_b2,
           mlp2_W1, mlp2_b1, mlp2_W2, mlp2_b2,
           lin_W, lin_b):
    n = x.shape[0]
    g = 64

    s0, d0 = edge_index_0[0], edge_index_0[1]
    s1, d1 = edge_index_1[0], edge_index_1[1]
    s2, d2 = edge_index_2[0], edge_index_2[1]

    w1cat = jnp.concatenate([c11_W1, c12_W1, c13_W1], axis=1)
    w1cat2 = jnp.concatenate([c21_W1, c22_W1, c23_W1], axis=1)
    b1s_1 = jnp.stack([c11_b1, c12_b1, c13_b1])
    w2s_1 = jnp.stack([c11_W2, c12_W2, c13_W2])
    b2s_1 = jnp.stack([c11_b2, c12_b2, c13_b2])
    b1s_2 = jnp.stack([c21_b1, c22_b1, c23_b1])
    w2s_2 = jnp.stack([c21_W2, c22_W2, c23_W2])
    b2s_2 = jnp.stack([c21_b2, c22_b2, c23_b2])

    y0, y1, y2 = _premul3(x, w1cat)
    p0, p1, p2 = _segsum3(y0, y1, y2, s0, d0, s1, d1, s2, d2)
    z0, z1, z2 = _mid(y0, y1, y2, p0, p1, p2, b1s_1, w2s_1, b2s_1,
                      mlp1_W1, mlp1_b1.reshape(1, 32), mlp1_W2,
                      mlp1_b2.reshape(1, 32), w1cat2)

    q0, q1, q2 = _segsum3(z0, z1, z2, s0, d0, s1, d1, s2, d2)
    batch3d = batch.reshape(n // BLK, 1, BLK)
    out = _final(z0, z1, z2, q0, q1, q2, batch3d, b1s_2, w2s_2, b2s_2,
                 mlp2_W1, mlp2_b1.reshape(1, 32), mlp2_W2,
                 mlp2_b2.reshape(1, 32), lin_W, lin_b.reshape(1, 1), g)
    return jnp.squeeze(out, axis=-1)


# speedup vs baseline: 13.6015x; 1.4541x over previous
import functools

import jax
import jax.numpy as jnp
from jax import lax
from jax.experimental import pallas as pl
from jax.experimental.pallas import tpu as pltpu
from jax.experimental.pallas import tpu_sc as plsc

BLK = 1000


def _mm_kernel(x_ref, w_ref, o0, o1, o2):
    y = jnp.dot(x_ref[...], w_ref[...], preferred_element_type=jnp.float32)
    o0[...] = y[:, 0:32]
    o1[...] = y[:, 32:64]
    o2[...] = y[:, 64:96]


def _premul3(x, w_cat):
    n, d = x.shape
    grid = n // BLK
    outs = [jax.ShapeDtypeStruct((n, 32), jnp.float32)] * 3
    return pl.pallas_call(
        _mm_kernel,
        grid=(grid,),
        in_specs=[
            pl.BlockSpec((BLK, d), lambda i: (i, 0)),
            pl.BlockSpec((d, 96), lambda i: (0, 0)),
        ],
        out_specs=[pl.BlockSpec((BLK, 32), lambda i: (i, 0))] * 3,
        out_shape=outs,
    )(x, w_cat)


def _segsum3(y0, y1, y2, s0, d0, s1, d1, s2, d2):
    n = y0.shape[0]
    e = s0.shape[0]
    info = plsc.get_sparse_core_info()
    nc, ns = info.num_cores, info.num_subcores
    nw = nc * ns
    epw = e // nw
    ch = 80
    nch = epw // ch
    assert ch * nch == epw and epw * nw == e
    rpt = (-(-n // ns) + 7) // 8 * 8
    n_pad = rpt * ns

    s0r, d0r = s0.reshape(nw, nch, ch), d0.reshape(nw, nch, ch)
    s1r, d1r = s1.reshape(nw, nch, ch), d1.reshape(nw, nch, ch)
    s2r, d2r = s2.reshape(nw, nch, ch), d2.reshape(nw, nch, ch)

    @functools.partial(
        pl.kernel,
        out_type=[jax.ShapeDtypeStruct((nc, n_pad, 32), jnp.float32)] * 3,
        mesh=plsc.VectorSubcoreMesh(core_axis_name="c", subcore_axis_name="s"),
        scratch_types=[
            pltpu.VMEM_SHARED((n_pad, 32), jnp.float32),
            pltpu.VMEM_SHARED((n_pad, 32), jnp.float32),
            pltpu.VMEM_SHARED((n_pad, 32), jnp.float32),
            [pltpu.VMEM((nch, ch), jnp.int32) for _ in range(6)],
            [pltpu.VMEM((ch, 32), jnp.float32) for _ in range(3)],
            [pltpu.SemaphoreType.DMA for _ in range(3)],
            [pltpu.SemaphoreType.DMA for _ in range(3)],
        ],
        compiler_params=pltpu.CompilerParams(use_tc_tiling_on_sc=False),
    )
    def k(y0h, y1h, y2h, s0h, d0h, s1h, d1h, s2h, d2h,
          o0, o1, o2, a0, a1, a2, idxs, rows, gsem, ssem):
        si0, di0, si1, di1, si2, di2 = idxs
        rows_a = rows[0]
        cid = lax.axis_index("c")
        sid = lax.axis_index("s")
        wid = sid * nc + cid

        for hb, vm in ((s0h, si0), (d0h, di0), (s1h, si1),
                       (d1h, di1), (s2h, si2), (d2h, di2)):
            pltpu.sync_copy(hb.at[wid], vm)

        zero16 = jnp.zeros((16,), jnp.float32)

        def zb(i, carry):
            rows_a[i, pl.ds(0, 16)] = zero16
            rows_a[i, pl.ds(16, 16)] = zero16
            return carry

        lax.fori_loop(0, ch, zb, 0)
        r0 = sid * rpt
        nfull, rem = divmod(rpt, ch)
        for a in (a0, a1, a2):
            for j in range(nfull):
                pltpu.sync_copy(rows_a, a.at[pl.ds(r0 + j * ch, ch)])
            if rem:
                pltpu.sync_copy(rows_a.at[pl.ds(0, rem)],
                                a.at[pl.ds(r0 + nfull * ch, rem)])
        plsc.subcore_barrier()

        nb = 3
        ngrp = (nch - 1) // nb
        for yh, si, di, a in ((y0h, si0, di0, a0),
                              (y1h, si1, di1, a1),
                              (y2h, si2, di2, a2)):
            for b in range(nb):
                pltpu.async_copy(yh.at[si.at[b]], rows[b], gsem[b])

            def grp(g, carry):
                for b in range(nb):
                    c = g * nb + b
                    pltpu.make_async_copy(
                        yh.at[si.at[c]], rows[b], gsem[b]).wait()
                    pltpu.async_copy(rows[b], a.at[di.at[c]], ssem[b],
                                     add=True)
                for b in range(nb):
                    c_next = g * nb + b + nb

                    @pl.when(c_next < nch)
                    def _():
                        pltpu.make_async_copy(
                            rows[b], a.at[di.at[0]], ssem[b]).wait()
                        pltpu.async_copy(
                            yh.at[si.at[c_next]], rows[b], gsem[b])
                return carry

            lax.fori_loop(0, ngrp, grp, 0)
            for b in range(nch - ngrp * nb):
                c = ngrp * nb + b
                pltpu.make_async_copy(yh.at[si.at[c]], rows[b], gsem[b]).wait()
                pltpu.async_copy(rows[b], a.at[di.at[c]], ssem[b], add=True)
            for b in range(nb):
                pltpu.make_async_copy(rows[b], a.at[di.at[0]], ssem[b]).wait()
        plsc.subcore_barrier()

        for a, o in ((a0, o0), (a1, o1), (a2, o2)):
            pltpu.sync_copy(a.at[pl.ds(r0, rpt)], o.at[cid, pl.ds(r0, rpt)])

    return k(y0, y1, y2, s0r, d0r, s1r, d1r, s2r, d2r)


def _mid_kernel(y0, y1, y2, p0, p1, p2, b1s, w2s, b2s,
                m1w1, m1b1, m1w2, m1b2, w1cat2, o0, o1, o2):
    ts = []
    for i, (y, p) in enumerate(((y0, p0), (y1, p1), (y2, p2))):
        pre = y[...] + p[0] + p[1] + b1s[pl.ds(i, 1)]
        t = jnp.dot(jnp.maximum(pre, 0.0), w2s[i],
                    preferred_element_type=jnp.float32) + b2s[pl.ds(i, 1)]
        ts.append(t)
    tcat = jnp.concatenate(ts, axis=1)
    u = jnp.maximum(jnp.dot(tcat, m1w1[...],
                            preferred_element_type=jnp.float32) + m1b1[...], 0.0)
    u = jnp.dot(u, m1w2[...], preferred_element_type=jnp.float32) + m1b2[...]
    z = jnp.dot(u, w1cat2[...], preferred_element_type=jnp.float32)
    o0[...] = z[:, 0:32]
    o1[...] = z[:, 32:64]
    o2[...] = z[:, 64:96]


def _mid(y0, y1, y2, p0, p1, p2, b1s, w2s, b2s, m1w1, m1b1, m1w2, m1b2, w1cat2):
    n = y0.shape[0]
    grid = n // BLK
    yspec = pl.BlockSpec((BLK, 32), lambda i: (i, 0))
    pspec = pl.BlockSpec((2, BLK, 32), lambda i: (0, i, 0))
    full = lambda s: pl.BlockSpec(s, lambda i: tuple(0 for _ in s))
    return pl.pallas_call(
        _mid_kernel,
        grid=(grid,),
        in_specs=[yspec] * 3 + [pspec] * 3 + [
            full((3, 32)), full((3, 32, 32)), full((3, 32)),
            full((96, 32)), full((1, 32)), full((32, 32)), full((1, 32)),
            full((32, 96)),
        ],
        out_specs=[yspec] * 3,
        out_shape=[jax.ShapeDtypeStruct((n, 32), jnp.float32)] * 3,
    )(y0, y1, y2, p0, p1, p2, b1s, w2s, b2s, m1w1, m1b1, m1w2, m1b2, w1cat2)


def _fin_kernel(z0, z1, z2, q0, q1, q2, batch_ref, b1s, w2s, b2s,
                m2w1, m2b1, m2w2, m2b2, linw, linb, out_ref, acc):
    i = pl.program_id(0)
    nblk = pl.num_programs(0)
    xs = []
    for k, (z, q) in enumerate(((z0, q0), (z1, q1), (z2, q2))):
        pre = z[...] + q[0] + q[1] + b1s[pl.ds(k, 1)]
        t = jnp.dot(jnp.maximum(pre, 0.0), w2s[k],
                    preferred_element_type=jnp.float32) + b2s[pl.ds(k, 1)]
        xs.append(jnp.maximum(t, 0.0))
    cat = jnp.concatenate(xs, axis=1)
    v = jnp.maximum(jnp.dot(cat, m2w1[...],
                            preferred_element_type=jnp.float32) + m2b1[...], 0.0)
    v = jnp.dot(v, m2w2[...], preferred_element_type=jnp.float32) + m2b2[...]
    bb = batch_ref[0]
    g = acc.shape[0]
    oh_t = jnp.where(
        jax.lax.broadcasted_iota(jnp.int32, (g, v.shape[0]), 0) == bb,
        1.0, 0.0)
    part = jnp.dot(oh_t, v, preferred_element_type=jnp.float32)

    @pl.when(i == 0)
    def _():
        acc[...] = jnp.zeros_like(acc)

    acc[...] += part

    @pl.when(i == nblk - 1)
    def _():
        out_ref[...] = jnp.dot(acc[...], linw[...],
                               preferred_element_type=jnp.float32) + linb[...]


def _final(z0, z1, z2, q0, q1, q2, batch3d, b1s, w2s, b2s,
           m2w1, m2b1, m2w2, m2b2, linw, linb, g):
    n = z0.shape[0]
    grid = n // BLK
    zspec = pl.BlockSpec((BLK, 32), lambda i: (i, 0))
    pspec = pl.BlockSpec((2, BLK, 32), lambda i: (0, i, 0))
    full = lambda s: pl.BlockSpec(s, lambda i: tuple(0 for _ in s))
    return pl.pallas_call(
        _fin_kernel,
        grid=(grid,),
        in_specs=[zspec] * 3 + [pspec] * 3 + [
            pl.BlockSpec((1, 1, BLK), lambda i: (i, 0, 0)),
            full((3, 32)), full((3, 32, 32)), full((3, 32)),
            full((96, 32)), full((1, 32)), full((32, 32)), full((1, 32)),
            full((32, 1)), full((1, 1)),
        ],
        out_specs=full((g, 1)),
        out_shape=jax.ShapeDtypeStruct((g, 1), jnp.float32),
        scratch_shapes=[pltpu.VMEM((g, 32), jnp.float32)],
    )(z0, z1, z2, q0, q1, q2, batch3d, b1s, w2s, b2s,
      m2w1, m2b1, m2w2, m2b2, linw, linb)


def kernel(x, edge_index_0, edge_index_1, edge_index_2, batch,
           c11_W1, c11_b1, c11_W2, c11_b2,
           c12_W1, c12_b1, c12_W2, c12_b2,
           c13_W1, c13_b1, c13_W2, c13_b2,
           c21_W1, c21_b1, c21_W2, c21_b2,
           c22_W1, c22_b1, c22_W2, c22_b2,
           c23_W1, c23_b1, c23_W2, c23_b2,
           mlp1_W1, mlp1_b1, mlp1_W2, mlp1_b2,
           mlp2_W1, mlp2_b1, mlp2_W2, mlp2_b2,
           lin_W, lin_b):
    n = x.shape[0]
    g = 64

    s0, d0 = edge_index_0[0], edge_index_0[1]
    s1, d1 = edge_index_1[0], edge_index_1[1]
    s2, d2 = edge_index_2[0], edge_index_2[1]

    w1cat = jnp.concatenate([c11_W1, c12_W1, c13_W1], axis=1)
    w1cat2 = jnp.concatenate([c21_W1, c22_W1, c23_W1], axis=1)
    b1s_1 = jnp.stack([c11_b1, c12_b1, c13_b1])
    w2s_1 = jnp.stack([c11_W2, c12_W2, c13_W2])
    b2s_1 = jnp.stack([c11_b2, c12_b2, c13_b2])
    b1s_2 = jnp.stack([c21_b1, c22_b1, c23_b1])
    w2s_2 = jnp.stack([c21_W2, c22_W2, c23_W2])
    b2s_2 = jnp.stack([c21_b2, c22_b2, c23_b2])

    y0, y1, y2 = _premul3(x, w1cat)
    p0, p1, p2 = _segsum3(y0, y1, y2, s0, d0, s1, d1, s2, d2)
    z0, z1, z2 = _mid(y0, y1, y2, p0, p1, p2, b1s_1, w2s_1, b2s_1,
                      mlp1_W1, mlp1_b1.reshape(1, 32), mlp1_W2,
                      mlp1_b2.reshape(1, 32), w1cat2)

    q0, q1, q2 = _segsum3(z0, z1, z2, s0, d0, s1, d1, s2, d2)
    batch3d = batch.reshape(n // BLK, 1, BLK)
    out = _final(z0, z1, z2, q0, q1, q2, batch3d, b1s_2, w2s_2, b2s_2,
                 mlp2_W1, mlp2_b1.reshape(1, 32), mlp2_W2,
                 mlp2_b2.reshape(1, 32), lin_W, lin_b.reshape(1, 1), g)
    return jnp.squeeze(out, axis=-1)


# speedup vs baseline: 14.1194x; 1.0381x over previous
import functools

import jax
import jax.numpy as jnp
from jax import lax
from jax.experimental import pallas as pl
from jax.experimental.pallas import tpu as pltpu
from jax.experimental.pallas import tpu_sc as plsc

BLK = 1000


def _mm_kernel(x_ref, w_ref, o0, o1, o2):
    y = jnp.dot(x_ref[...], w_ref[...], preferred_element_type=jnp.float32)
    o0[...] = y[:, 0:32]
    o1[...] = y[:, 32:64]
    o2[...] = y[:, 64:96]


def _premul3(x, w_cat):
    n, d = x.shape
    grid = n // BLK
    outs = [jax.ShapeDtypeStruct((n, 32), jnp.float32)] * 3
    return pl.pallas_call(
        _mm_kernel,
        grid=(grid,),
        in_specs=[
            pl.BlockSpec((BLK, d), lambda i: (i, 0)),
            pl.BlockSpec((d, 96), lambda i: (0, 0)),
        ],
        out_specs=[pl.BlockSpec((BLK, 32), lambda i: (i, 0))] * 3,
        out_shape=outs,
    )(x, w_cat)


def _segsum3(y0, y1, y2, s0, d0, s1, d1, s2, d2):
    n = y0.shape[0]
    e = s0.shape[0]
    info = plsc.get_sparse_core_info()
    nc, ns = info.num_cores, info.num_subcores
    nw = nc * ns
    epw = e // nw
    ch = 80
    nch = epw // ch
    nb = 5
    assert ch * nch == epw and epw * nw == e
    rpt = (-(-n // ns) + 7) // 8 * 8
    n_pad = rpt * ns

    s0r, d0r = s0.reshape(nw, nch, ch), d0.reshape(nw, nch, ch)
    s1r, d1r = s1.reshape(nw, nch, ch), d1.reshape(nw, nch, ch)
    s2r, d2r = s2.reshape(nw, nch, ch), d2.reshape(nw, nch, ch)

    @functools.partial(
        pl.kernel,
        out_type=[jax.ShapeDtypeStruct((nc, n_pad, 32), jnp.float32)] * 3,
        mesh=plsc.VectorSubcoreMesh(core_axis_name="c", subcore_axis_name="s"),
        scratch_types=[
            pltpu.VMEM_SHARED((n_pad, 32), jnp.float32),
            pltpu.VMEM_SHARED((n_pad, 32), jnp.float32),
            pltpu.VMEM_SHARED((n_pad, 32), jnp.float32),
            [pltpu.VMEM((ch,), jnp.int32) for _ in range(nb)],
            [pltpu.VMEM((ch,), jnp.int32) for _ in range(nb)],
            [pltpu.VMEM((ch, 32), jnp.float32) for _ in range(nb)],
            [pltpu.SemaphoreType.DMA for _ in range(nb)],
            [pltpu.SemaphoreType.DMA for _ in range(nb)],
            [pltpu.SemaphoreType.DMA for _ in range(nb)],
        ],
        compiler_params=pltpu.CompilerParams(use_tc_tiling_on_sc=False),
    )
    def k(y0h, y1h, y2h, s0h, d0h, s1h, d1h, s2h, d2h,
          o0, o1, o2, a0, a1, a2, sidx, didx, rows, isem, gsem, ssem):
        cid = lax.axis_index("c")
        sid = lax.axis_index("s")
        wid = sid * nc + cid
        rows_a = rows[0]

        zero16 = jnp.zeros((16,), jnp.float32)

        def zb(i, carry):
            rows_a[i, pl.ds(0, 16)] = zero16
            rows_a[i, pl.ds(16, 16)] = zero16
            return carry

        lax.fori_loop(0, ch, zb, 0)
        r0 = sid * rpt
        nfull, rem = divmod(rpt, ch)
        for a in (a0, a1, a2):
            for j in range(nfull):
                pltpu.sync_copy(rows_a, a.at[pl.ds(r0 + j * ch, ch)])
            if rem:
                pltpu.sync_copy(rows_a.at[pl.ds(0, rem)],
                                a.at[pl.ds(r0 + nfull * ch, rem)])
        plsc.subcore_barrier()

        ngrp = nch // nb
        npeel = nch - ngrp * nb
        for yh, sh, dh, a in ((y0h, s0h, d0h, a0),
                              (y1h, s1h, d1h, a1),
                              (y2h, s2h, d2h, a2)):
            for b in range(nb):
                pltpu.async_copy(sh.at[wid, b], sidx[b], isem[b])
                pltpu.async_copy(dh.at[wid, b], didx[b], isem[b])

            def grp(g, carry):
                for b in range(nb):
                    c = g * nb + b
                    pltpu.make_async_copy(sh.at[wid, c], sidx[b],
                                          isem[b]).wait()
                    pltpu.make_async_copy(dh.at[wid, c], didx[b],
                                          isem[b]).wait()
                    pltpu.async_copy(yh.at[sidx[b]], rows[b], gsem[b])
                for b in range(nb):
                    c = g * nb + b
                    pltpu.make_async_copy(yh.at[sidx[b]], rows[b],
                                          gsem[b]).wait()
                    pltpu.async_copy(rows[b], a.at[didx[b]], ssem[b],
                                     add=True)
                for b in range(nb):
                    c_next = g * nb + b + nb

                    @pl.when(c_next < nch)
                    def _():
                        pltpu.make_async_copy(rows[b], a.at[didx[b]],
                                              ssem[b]).wait()
                        pltpu.async_copy(sh.at[wid, c_next], sidx[b], isem[b])
                        pltpu.async_copy(dh.at[wid, c_next], didx[b], isem[b])
                return carry

            lax.fori_loop(0, ngrp, grp, 0)
            for b in range(npeel):
                c = ngrp * nb + b
                pltpu.make_async_copy(sh.at[wid, c], sidx[b], isem[b]).wait()
                pltpu.make_async_copy(dh.at[wid, c], didx[b], isem[b]).wait()
                pltpu.async_copy(yh.at[sidx[b]], rows[b], gsem[b])
            for b in range(npeel):
                pltpu.make_async_copy(yh.at[sidx[b]], rows[b], gsem[b]).wait()
                pltpu.async_copy(rows[b], a.at[didx[b]], ssem[b], add=True)
            for b in range(nb):
                pltpu.make_async_copy(rows[b], a.at[didx[b]], ssem[b]).wait()
        plsc.subcore_barrier()

        for a, o in ((a0, o0), (a1, o1), (a2, o2)):
            pltpu.sync_copy(a.at[pl.ds(r0, rpt)], o.at[cid, pl.ds(r0, rpt)])

    return k(y0, y1, y2, s0r, d0r, s1r, d1r, s2r, d2r)


def _mid_kernel(y0, y1, y2, p0, p1, p2, b1s, w2s, b2s,
                m1w1, m1b1, m1w2, m1b2, w1cat2, o0, o1, o2):
    ts = []
    for i, (y, p) in enumerate(((y0, p0), (y1, p1), (y2, p2))):
        pre = y[...] + p[0] + p[1] + b1s[pl.ds(i, 1)]
        t = jnp.dot(jnp.maximum(pre, 0.0), w2s[i],
                    preferred_element_type=jnp.float32) + b2s[pl.ds(i, 1)]
        ts.append(t)
    tcat = jnp.concatenate(ts, axis=1)
    u = jnp.maximum(jnp.dot(tcat, m1w1[...],
                            preferred_element_type=jnp.float32) + m1b1[...], 0.0)
    u = jnp.dot(u, m1w2[...], preferred_element_type=jnp.float32) + m1b2[...]
    z = jnp.dot(u, w1cat2[...], preferred_element_type=jnp.float32)
    o0[...] = z[:, 0:32]
    o1[...] = z[:, 32:64]
    o2[...] = z[:, 64:96]


def _mid(y0, y1, y2, p0, p1, p2, b1s, w2s, b2s, m1w1, m1b1, m1w2, m1b2, w1cat2):
    n = y0.shape[0]
    grid = n // BLK
    yspec = pl.BlockSpec((BLK, 32), lambda i: (i, 0))
    pspec = pl.BlockSpec((2, BLK, 32), lambda i: (0, i, 0))
    full = lambda s: pl.BlockSpec(s, lambda i: tuple(0 for _ in s))
    return pl.pallas_call(
        _mid_kernel,
        grid=(grid,),
        in_specs=[yspec] * 3 + [pspec] * 3 + [
            full((3, 32)), full((3, 32, 32)), full((3, 32)),
            full((96, 32)), full((1, 32)), full((32, 32)), full((1, 32)),
            full((32, 96)),
        ],
        out_specs=[yspec] * 3,
        out_shape=[jax.ShapeDtypeStruct((n, 32), jnp.float32)] * 3,
    )(y0, y1, y2, p0, p1, p2, b1s, w2s, b2s, m1w1, m1b1, m1w2, m1b2, w1cat2)


def _fin_kernel(z0, z1, z2, q0, q1, q2, batch_ref, b1s, w2s, b2s,
                m2w1, m2b1, m2w2, m2b2, linw, linb, out_ref, acc):
    i = pl.program_id(0)
    nblk = pl.num_programs(0)
    xs = []
    for k, (z, q) in enumerate(((z0, q0), (z1, q1), (z2, q2))):
        pre = z[...] + q[0] + q[1] + b1s[pl.ds(k, 1)]
        t = jnp.dot(jnp.maximum(pre, 0.0), w2s[k],
                    preferred_element_type=jnp.float32) + b2s[pl.ds(k, 1)]
        xs.append(jnp.maximum(t, 0.0))
    cat = jnp.concatenate(xs, axis=1)
    v = jnp.maximum(jnp.dot(cat, m2w1[...],
                            preferred_element_type=jnp.float32) + m2b1[...], 0.0)
    v = jnp.dot(v, m2w2[...], preferred_element_type=jnp.float32) + m2b2[...]
    bb = batch_ref[0]
    g = acc.shape[0]
    oh_t = jnp.where(
        jax.lax.broadcasted_iota(jnp.int32, (g, v.shape[0]), 0) == bb,
        1.0, 0.0)
    part = jnp.dot(oh_t, v, preferred_element_type=jnp.float32)

    @pl.when(i == 0)
    def _():
        acc[...] = jnp.zeros_like(acc)

    acc[...] += part

    @pl.when(i == nblk - 1)
    def _():
        out_ref[...] = jnp.dot(acc[...], linw[...],
                               preferred_element_type=jnp.float32) + linb[...]


def _final(z0, z1, z2, q0, q1, q2, batch3d, b1s, w2s, b2s,
           m2w1, m2b1, m2w2, m2b2, linw, linb, g):
    n = z0.shape[0]
    grid = n // BLK
    zspec = pl.BlockSpec((BLK, 32), lambda i: (i, 0))
    pspec = pl.BlockSpec((2, BLK, 32), lambda i: (0, i, 0))
    full = lambda s: pl.BlockSpec(s, lambda i: tuple(0 for _ in s))
    return pl.pallas_call(
        _fin_kernel,
        grid=(grid,),
        in_specs=[zspec] * 3 + [pspec] * 3 + [
            pl.BlockSpec((1, 1, BLK), lambda i: (i, 0, 0)),
            full((3, 32)), full((3, 32, 32)), full((3, 32)),
            full((96, 32)), full((1, 32)), full((32, 32)), full((1, 32)),
            full((32, 1)), full((1, 1)),
        ],
        out_specs=full((g, 1)),
        out_shape=jax.ShapeDtypeStruct((g, 1), jnp.float32),
        scratch_shapes=[pltpu.VMEM((g, 32), jnp.float32)],
    )(z0, z1, z2, q0, q1, q2, batch3d, b1s, w2s, b2s,
      m2w1, m2b1, m2w2, m2b2, linw, linb)


def kernel(x, edge_index_0, edge_index_1, edge_index_2, batch,
           c11_W1, c11_b1, c11_W2, c11_b2,
           c12_W1, c12_b1, c12_W2, c12_b2,
           c13_W1, c13_b1, c13_W2, c13_b2,
           c21_W1, c21_b1, c21_W2, c21_b2,
           c22_W1, c22_b1, c22_W2, c22_b2,
           c23_W1, c23_b1, c23_W2, c23_b2,
           mlp1_W1, mlp1_b1, mlp1_W2, mlp1_b2,
           mlp2_W1, mlp2_b1, mlp2_W2, mlp2_b2,
           lin_W, lin_b):
    n = x.shape[0]
    g = 64

    s0, d0 = edge_index_0[0], edge_index_0[1]
    s1, d1 = edge_index_1[0], edge_index_1[1]
    s2, d2 = edge_index_2[0], edge_index_2[1]

    w1cat = jnp.concatenate([c11_W1, c12_W1, c13_W1], axis=1)
    w1cat2 = jnp.concatenate([c21_W1, c22_W1, c23_W1], axis=1)
    b1s_1 = jnp.stack([c11_b1, c12_b1, c13_b1])
    w2s_1 = jnp.stack([c11_W2, c12_W2, c13_W2])
    b2s_1 = jnp.stack([c11_b2, c12_b2, c13_b2])
    b1s_2 = jnp.stack([c21_b1, c22_b1, c23_b1])
    w2s_2 = jnp.stack([c21_W2, c22_W2, c23_W2])
    b2s_2 = jnp.stack([c21_b2, c22_b2, c23_b2])

    y0, y1, y2 = _premul3(x, w1cat)
    p0, p1, p2 = _segsum3(y0, y1, y2, s0, d0, s1, d1, s2, d2)
    z0, z1, z2 = _mid(y0, y1, y2, p0, p1, p2, b1s_1, w2s_1, b2s_1,
                      mlp1_W1, mlp1_b1.reshape(1, 32), mlp1_W2,
                      mlp1_b2.reshape(1, 32), w1cat2)

    q0, q1, q2 = _segsum3(z0, z1, z2, s0, d0, s1, d1, s2, d2)
    batch3d = batch.reshape(n // BLK, 1, BLK)
    out = _final(z0, z1, z2, q0, q1, q2, batch3d, b1s_2, w2s_2, b2s_2,
                 mlp2_W1, mlp2_b1.reshape(1, 32), mlp2_W2,
                 mlp2_b2.reshape(1, 32), lin_W, lin_b.reshape(1, 1), g)
    return jnp.squeeze(out, axis=-1)


# speedup vs baseline: 15.8751x; 1.1243x over previous
import functools

import jax
import jax.numpy as jnp
from jax import lax
from jax.experimental import pallas as pl
from jax.experimental.pallas import tpu as pltpu
from jax.experimental.pallas import tpu_sc as plsc

BLK = 1000


def _mm_kernel(x_ref, w_ref, o0, o1, o2):
    y = jnp.dot(x_ref[...], w_ref[...], preferred_element_type=jnp.float32)
    o0[...] = y[:, 0:32]
    o1[...] = y[:, 32:64]
    o2[...] = y[:, 64:96]


def _premul3(x, w_cat):
    n, d = x.shape
    grid = n // BLK
    outs = [jax.ShapeDtypeStruct((n, 32), jnp.float32)] * 3
    return pl.pallas_call(
        _mm_kernel,
        grid=(grid,),
        in_specs=[
            pl.BlockSpec((BLK, d), lambda i: (i, 0)),
            pl.BlockSpec((d, 96), lambda i: (0, 0)),
        ],
        out_specs=[pl.BlockSpec((BLK, 32), lambda i: (i, 0))] * 3,
        out_shape=outs,
    )(x, w_cat)


def _segsum3(y0, y1, y2, s0, d0, s1, d1, s2, d2):
    n = y0.shape[0]
    e = s0.shape[0]
    info = plsc.get_sparse_core_info()
    nc, ns = info.num_cores, info.num_subcores
    nw = nc * ns
    epw = e // nw
    ch = 80
    nch = epw // ch
    nb = 8
    assert ch * nch == epw and epw * nw == e
    rpt = (-(-n // ns) + 7) // 8 * 8
    n_pad = rpt * ns

    s0r, d0r = s0.reshape(nw, nch, ch), d0.reshape(nw, nch, ch)
    s1r, d1r = s1.reshape(nw, nch, ch), d1.reshape(nw, nch, ch)
    s2r, d2r = s2.reshape(nw, nch, ch), d2.reshape(nw, nch, ch)

    @functools.partial(
        pl.kernel,
        out_type=[jax.ShapeDtypeStruct((nc, n_pad, 32), jnp.float32)] * 3,
        mesh=plsc.VectorSubcoreMesh(core_axis_name="c", subcore_axis_name="s"),
        scratch_types=[
            pltpu.VMEM_SHARED((n_pad, 32), jnp.float32),
            pltpu.VMEM_SHARED((n_pad, 32), jnp.float32),
            pltpu.VMEM_SHARED((n_pad, 32), jnp.float32),
            [pltpu.VMEM((ch,), jnp.int32) for _ in range(nb)],
            [pltpu.VMEM((ch,), jnp.int32) for _ in range(nb)],
            [pltpu.VMEM((ch, 32), jnp.float32) for _ in range(nb)],
            [pltpu.SemaphoreType.DMA for _ in range(nb)],
            [pltpu.SemaphoreType.DMA for _ in range(nb)],
            [pltpu.SemaphoreType.DMA for _ in range(nb)],
        ],
        compiler_params=pltpu.CompilerParams(use_tc_tiling_on_sc=False),
    )
    def k(y0h, y1h, y2h, s0h, d0h, s1h, d1h, s2h, d2h,
          o0, o1, o2, a0, a1, a2, sidx, didx, rows, isem, gsem, ssem):
        cid = lax.axis_index("c")
        sid = lax.axis_index("s")
        wid = sid * nc + cid
        rows_a = rows[0]

        zero16 = jnp.zeros((16,), jnp.float32)

        def zb(i, carry):
            rows_a[i, pl.ds(0, 16)] = zero16
            rows_a[i, pl.ds(16, 16)] = zero16
            return carry

        lax.fori_loop(0, ch, zb, 0)
        r0 = sid * rpt
        nfull, rem = divmod(rpt, ch)
        for a in (a0, a1, a2):
            for j in range(nfull):
                pltpu.sync_copy(rows_a, a.at[pl.ds(r0 + j * ch, ch)])
            if rem:
                pltpu.sync_copy(rows_a.at[pl.ds(0, rem)],
                                a.at[pl.ds(r0 + nfull * ch, rem)])
        plsc.subcore_barrier()

        ngrp = nch // nb
        npeel = nch - ngrp * nb
        for yh, sh, dh, a in ((y0h, s0h, d0h, a0),
                              (y1h, s1h, d1h, a1),
                              (y2h, s2h, d2h, a2)):
            for b in range(nb):
                pltpu.async_copy(sh.at[wid, b], sidx[b], isem[b])
                pltpu.async_copy(dh.at[wid, b], didx[b], isem[b])

            def grp(g, carry):
                for b in range(nb):
                    c = g * nb + b
                    pltpu.make_async_copy(sh.at[wid, c], sidx[b],
                                          isem[b]).wait()
                    pltpu.make_async_copy(dh.at[wid, c], didx[b],
                                          isem[b]).wait()
                    pltpu.async_copy(yh.at[sidx[b]], rows[b], gsem[b])
                for b in range(nb):
                    c = g * nb + b
                    pltpu.make_async_copy(yh.at[sidx[b]], rows[b],
                                          gsem[b]).wait()
                    pltpu.async_copy(rows[b], a.at[didx[b]], ssem[b],
                                     add=True)
                for b in range(nb):
                    c_next = g * nb + b + nb

                    @pl.when(c_next < nch)
                    def _():
                        pltpu.make_async_copy(rows[b], a.at[didx[b]],
                                              ssem[b]).wait()
                        pltpu.async_copy(sh.at[wid, c_next], sidx[b], isem[b])
                        pltpu.async_copy(dh.at[wid, c_next], didx[b], isem[b])
                return carry

            lax.fori_loop(0, ngrp, grp, 0)
            for b in range(npeel):
                c = ngrp * nb + b
                pltpu.make_async_copy(sh.at[wid, c], sidx[b], isem[b]).wait()
                pltpu.make_async_copy(dh.at[wid, c], didx[b], isem[b]).wait()
                pltpu.async_copy(yh.at[sidx[b]], rows[b], gsem[b])
            for b in range(npeel):
                pltpu.make_async_copy(yh.at[sidx[b]], rows[b], gsem[b]).wait()
                pltpu.async_copy(rows[b], a.at[didx[b]], ssem[b], add=True)
            for b in range(nb):
                pltpu.make_async_copy(rows[b], a.at[didx[b]], ssem[b]).wait()
        plsc.subcore_barrier()

        for a, o in ((a0, o0), (a1, o1), (a2, o2)):
            pltpu.sync_copy(a.at[pl.ds(r0, rpt)], o.at[cid, pl.ds(r0, rpt)])

    return k(y0, y1, y2, s0r, d0r, s1r, d1r, s2r, d2r)


def _mid_kernel(y0, y1, y2, p0, p1, p2, b1s, w2s, b2s,
                m1w1, m1b1, m1w2, m1b2, w1cat2, o0, o1, o2):
    ts = []
    for i, (y, p) in enumerate(((y0, p0), (y1, p1), (y2, p2))):
        pre = y[...] + p[0] + p[1] + b1s[pl.ds(i, 1)]
        t = jnp.dot(jnp.maximum(pre, 0.0), w2s[i],
                    preferred_element_type=jnp.float32) + b2s[pl.ds(i, 1)]
        ts.append(t)
    tcat = jnp.concatenate(ts, axis=1)
    u = jnp.maximum(jnp.dot(tcat, m1w1[...],
                            preferred_element_type=jnp.float32) + m1b1[...], 0.0)
    u = jnp.dot(u, m1w2[...], preferred_element_type=jnp.float32) + m1b2[...]
    z = jnp.dot(u, w1cat2[...], preferred_element_type=jnp.float32)
    o0[...] = z[:, 0:32]
    o1[...] = z[:, 32:64]
    o2[...] = z[:, 64:96]


def _mid(y0, y1, y2, p0, p1, p2, b1s, w2s, b2s, m1w1, m1b1, m1w2, m1b2, w1cat2):
    n = y0.shape[0]
    grid = n // BLK
    yspec = pl.BlockSpec((BLK, 32), lambda i: (i, 0))
    pspec = pl.BlockSpec((2, BLK, 32), lambda i: (0, i, 0))
    full = lambda s: pl.BlockSpec(s, lambda i: tuple(0 for _ in s))
    return pl.pallas_call(
        _mid_kernel,
        grid=(grid,),
        in_specs=[yspec] * 3 + [pspec] * 3 + [
            full((3, 32)), full((3, 32, 32)), full((3, 32)),
            full((96, 32)), full((1, 32)), full((32, 32)), full((1, 32)),
            full((32, 96)),
        ],
        out_specs=[yspec] * 3,
        out_shape=[jax.ShapeDtypeStruct((n, 32), jnp.float32)] * 3,
    )(y0, y1, y2, p0, p1, p2, b1s, w2s, b2s, m1w1, m1b1, m1w2, m1b2, w1cat2)


def _fin_kernel(z0, z1, z2, q0, q1, q2, batch_ref, b1s, w2s, b2s,
                m2w1, m2b1, m2w2, m2b2, linw, linb, out_ref, acc):
    i = pl.program_id(0)
    nblk = pl.num_programs(0)
    xs = []
    for k, (z, q) in enumerate(((z0, q0), (z1, q1), (z2, q2))):
        pre = z[...] + q[0] + q[1] + b1s[pl.ds(k, 1)]
        t = jnp.dot(jnp.maximum(pre, 0.0), w2s[k],
                    preferred_element_type=jnp.float32) + b2s[pl.ds(k, 1)]
        xs.append(jnp.maximum(t, 0.0))
    cat = jnp.concatenate(xs, axis=1)
    v = jnp.maximum(jnp.dot(cat, m2w1[...],
                            preferred_element_type=jnp.float32) + m2b1[...], 0.0)
    v = jnp.dot(v, m2w2[...], preferred_element_type=jnp.float32) + m2b2[...]
    bb = batch_ref[0]
    g = acc.shape[0]
    oh_t = jnp.where(
        jax.lax.broadcasted_iota(jnp.int32, (g, v.shape[0]), 0) == bb,
        1.0, 0.0)
    part = jnp.dot(oh_t, v, preferred_element_type=jnp.float32)

    @pl.when(i == 0)
    def _():
        acc[...] = jnp.zeros_like(acc)

    acc[...] += part

    @pl.when(i == nblk - 1)
    def _():
        out_ref[...] = jnp.dot(acc[...], linw[...],
                               preferred_element_type=jnp.float32) + linb[...]


def _final(z0, z1, z2, q0, q1, q2, batch3d, b1s, w2s, b2s,
           m2w1, m2b1, m2w2, m2b2, linw, linb, g):
    n = z0.shape[0]
    grid = n // BLK
    zspec = pl.BlockSpec((BLK, 32), lambda i: (i, 0))
    pspec = pl.BlockSpec((2, BLK, 32), lambda i: (0, i, 0))
    full = lambda s: pl.BlockSpec(s, lambda i: tuple(0 for _ in s))
    return pl.pallas_call(
        _fin_kernel,
        grid=(grid,),
        in_specs=[zspec] * 3 + [pspec] * 3 + [
            pl.BlockSpec((1, 1, BLK), lambda i: (i, 0, 0)),
            full((3, 32)), full((3, 32, 32)), full((3, 32)),
            full((96, 32)), full((1, 32)), full((32, 32)), full((1, 32)),
            full((32, 1)), full((1, 1)),
        ],
        out_specs=full((g, 1)),
        out_shape=jax.ShapeDtypeStruct((g, 1), jnp.float32),
        scratch_shapes=[pltpu.VMEM((g, 32), jnp.float32)],
    )(z0, z1, z2, q0, q1, q2, batch3d, b1s, w2s, b2s,
      m2w1, m2b1, m2w2, m2b2, linw, linb)


def kernel(x, edge_index_0, edge_index_1, edge_index_2, batch,
           c11_W1, c11_b1, c11_W2, c11_b2,
           c12_W1, c12_b1, c12_W2, c12_b2,
           c13_W1, c13_b1, c13_W2, c13_b2,
           c21_W1, c21_b1, c21_W2, c21_b2,
           c22_W1, c22_b1, c22_W2, c22_b2,
           c23_W1, c23_b1, c23_W2, c23_b2,
           mlp1_W1, mlp1_b1, mlp1_W2, mlp1_b2,
           mlp2_W1, mlp2_b1, mlp2_W2, mlp2_b2,
           lin_W, lin_b):
    n = x.shape[0]
    g = 64

    s0, d0 = edge_index_0[0], edge_index_0[1]
    s1, d1 = edge_index_1[0], edge_index_1[1]
    s2, d2 = edge_index_2[0], edge_index_2[1]

    w1cat = jnp.concatenate([c11_W1, c12_W1, c13_W1], axis=1)
    w1cat2 = jnp.concatenate([c21_W1, c22_W1, c23_W1], axis=1)
    b1s_1 = jnp.stack([c11_b1, c12_b1, c13_b1])
    w2s_1 = jnp.stack([c11_W2, c12_W2, c13_W2])
    b2s_1 = jnp.stack([c11_b2, c12_b2, c13_b2])
    b1s_2 = jnp.stack([c21_b1, c22_b1, c23_b1])
    w2s_2 = jnp.stack([c21_W2, c22_W2, c23_W2])
    b2s_2 = jnp.stack([c21_b2, c22_b2, c23_b2])

    y0, y1, y2 = _premul3(x, w1cat)
    p0, p1, p2 = _segsum3(y0, y1, y2, s0, d0, s1, d1, s2, d2)
    z0, z1, z2 = _mid(y0, y1, y2, p0, p1, p2, b1s_1, w2s_1, b2s_1,
                      mlp1_W1, mlp1_b1.reshape(1, 32), mlp1_W2,
                      mlp1_b2.reshape(1, 32), w1cat2)

    q0, q1, q2 = _segsum3(z0, z1, z2, s0, d0, s1, d1, s2, d2)
    batch3d = batch.reshape(n // BLK, 1, BLK)
    out = _final(z0, z1, z2, q0, q1, q2, batch3d, b1s_2, w2s_2, b2s_2,
                 mlp2_W1, mlp2_b1.reshape(1, 32), mlp2_W2,
                 mlp2_b2.reshape(1, 32), lin_W, lin_b.reshape(1, 1), g)
    return jnp.squeeze(out, axis=-1)


# speedup vs baseline: 16.9995x; 1.0708x over previous
import functools

import jax
import jax.numpy as jnp
from jax import lax
from jax.experimental import pallas as pl
from jax.experimental.pallas import tpu as pltpu
from jax.experimental.pallas import tpu_sc as plsc

BLK = 1000


def _mm_kernel(x_ref, w_ref, o0, o1, o2):
    y = jnp.dot(x_ref[...], w_ref[...], preferred_element_type=jnp.float32)
    o0[...] = y[:, 0:32]
    o1[...] = y[:, 32:64]
    o2[...] = y[:, 64:96]


def _premul3(x, w_cat):
    n, d = x.shape
    grid = n // BLK
    outs = [jax.ShapeDtypeStruct((n, 32), jnp.float32)] * 3
    return pl.pallas_call(
        _mm_kernel,
        grid=(grid,),
        in_specs=[
            pl.BlockSpec((BLK, d), lambda i: (i, 0)),
            pl.BlockSpec((d, 96), lambda i: (0, 0)),
        ],
        out_specs=[pl.BlockSpec((BLK, 32), lambda i: (i, 0))] * 3,
        out_shape=outs,
    )(x, w_cat)


def _segsum3(y0, y1, y2, e0, e1, e2):
    n = y0.shape[0]
    e = e0.shape[1]
    info = plsc.get_sparse_core_info()
    nc, ns = info.num_cores, info.num_subcores
    nw = nc * ns
    epw = e // nw
    ch = 80
    nch = epw // ch
    nb = 8
    assert ch * nch == epw and epw * nw == e
    rpt = (-(-n // ns) + 7) // 8 * 8
    n_pad = rpt * ns

    @functools.partial(
        pl.kernel,
        out_type=[jax.ShapeDtypeStruct((n_pad, 32), jnp.float32)] * 6,
        mesh=plsc.VectorSubcoreMesh(core_axis_name="c", subcore_axis_name="s"),
        scratch_types=[
            pltpu.VMEM_SHARED((n_pad, 32), jnp.float32),
            pltpu.VMEM_SHARED((n_pad, 32), jnp.float32),
            pltpu.VMEM_SHARED((n_pad, 32), jnp.float32),
            [pltpu.VMEM((ch,), jnp.int32) for _ in range(nb)],
            [pltpu.VMEM((ch,), jnp.int32) for _ in range(nb)],
            [pltpu.VMEM((ch, 32), jnp.float32) for _ in range(nb)],
            [pltpu.SemaphoreType.DMA for _ in range(nb)],
            [pltpu.SemaphoreType.DMA for _ in range(nb)],
            [pltpu.SemaphoreType.DMA for _ in range(nb)],
        ],
        compiler_params=pltpu.CompilerParams(use_tc_tiling_on_sc=False),
    )
    def k(y0h, y1h, y2h, e0h, e1h, e2h,
          o00, o01, o10, o11, o20, o21,
          a0, a1, a2, sidx, didx, rows, isem, gsem, ssem):
        cid = lax.axis_index("c")
        sid = lax.axis_index("s")
        wid = sid * nc + cid
        base = wid * epw
        rows_a = rows[0]

        def prefetch(eh, c, b):
            pltpu.async_copy(eh.at[0, pl.ds(base + c * ch, ch)],
                             sidx[b], isem[b])
            pltpu.async_copy(eh.at[1, pl.ds(base + c * ch, ch)],
                             didx[b], isem[b])

        def wait_prefetch(eh, c, b):
            pltpu.make_async_copy(eh.at[0, pl.ds(base + c * ch, ch)],
                                  sidx[b], isem[b]).wait()
            pltpu.make_async_copy(eh.at[1, pl.ds(base + c * ch, ch)],
                                  didx[b], isem[b]).wait()

        for b in range(nb):
            prefetch(e0h, b, b)

        zero16 = jnp.zeros((16,), jnp.float32)

        def zb(i, carry):
            rows_a[i, pl.ds(0, 16)] = zero16
            rows_a[i, pl.ds(16, 16)] = zero16
            return carry

        lax.fori_loop(0, ch, zb, 0)
        r0 = sid * rpt
        nfull, rem = divmod(rpt, ch)
        for a in (a0, a1, a2):
            for j in range(nfull):
                pltpu.sync_copy(rows_a, a.at[pl.ds(r0 + j * ch, ch)])
            if rem:
                pltpu.sync_copy(rows_a.at[pl.ds(0, rem)],
                                a.at[pl.ds(r0 + nfull * ch, rem)])
        plsc.subcore_barrier()

        ngrp = nch // nb
        npeel = nch - ngrp * nb
        sets = ((y0h, e0h, a0), (y1h, e1h, a1), (y2h, e2h, a2))
        for ksi, (yh, eh, a) in enumerate(sets):
            if ksi > 0:
                for b in range(nb):
                    prefetch(eh, b, b)

            def grp(g, carry):
                for b in range(nb):
                    c = g * nb + b
                    wait_prefetch(eh, c, b)
                    pltpu.async_copy(yh.at[sidx[b]], rows[b], gsem[b])
                for b in range(nb):
                    pltpu.make_async_copy(yh.at[sidx[b]], rows[b],
                                          gsem[b]).wait()
                    pltpu.async_copy(rows[b], a.at[didx[b]], ssem[b],
                                     add=True)
                for b in range(nb):
                    c_next = g * nb + b + nb

                    @pl.when(c_next < nch)
                    def _():
                        pltpu.make_async_copy(rows[b], a.at[didx[b]],
                                              ssem[b]).wait()
                        prefetch(eh, c_next, b)
                return carry

            lax.fori_loop(0, ngrp, grp, 0)
            for b in range(npeel):
                c = ngrp * nb + b
                wait_prefetch(eh, c, b)
                pltpu.async_copy(yh.at[sidx[b]], rows[b], gsem[b])
            for b in range(npeel):
                pltpu.make_async_copy(yh.at[sidx[b]], rows[b], gsem[b]).wait()
                pltpu.async_copy(rows[b], a.at[didx[b]], ssem[b], add=True)
            for b in range(nb):
                pltpu.make_async_copy(rows[b], a.at[didx[b]], ssem[b]).wait()
        plsc.subcore_barrier()

        for a, oc0, oc1 in ((a0, o00, o01), (a1, o10, o11), (a2, o20, o21)):
            @pl.when(cid == 0)
            def _():
                pltpu.sync_copy(a.at[pl.ds(r0, rpt)], oc0.at[pl.ds(r0, rpt)])

            @pl.when(cid == 1)
            def _():
                pltpu.sync_copy(a.at[pl.ds(r0, rpt)], oc1.at[pl.ds(r0, rpt)])

    return k(y0, y1, y2, e0, e1, e2)


def _mid_kernel(y0, y1, y2, p00, p01, p10, p11, p20, p21, b1s, w2s, b2s,
                m1w1, m1b1, m1w2, m1b2, w1cat2, o0, o1, o2):
    ts = []
    for i, (y, pa, pb) in enumerate(((y0, p00, p01), (y1, p10, p11),
                                     (y2, p20, p21))):
        pre = y[...] + pa[...] + pb[...] + b1s[pl.ds(i, 1)]
        t = jnp.dot(jnp.maximum(pre, 0.0), w2s[i],
                    preferred_element_type=jnp.float32) + b2s[pl.ds(i, 1)]
        ts.append(t)
    tcat = jnp.concatenate(ts, axis=1)
    u = jnp.maximum(jnp.dot(tcat, m1w1[...],
                            preferred_element_type=jnp.float32) + m1b1[...], 0.0)
    u = jnp.dot(u, m1w2[...], preferred_element_type=jnp.float32) + m1b2[...]
    z = jnp.dot(u, w1cat2[...], preferred_element_type=jnp.float32)
    o0[...] = z[:, 0:32]
    o1[...] = z[:, 32:64]
    o2[...] = z[:, 64:96]


def _mid(y0, y1, y2, ps, b1s, w2s, b2s, m1w1, m1b1, m1w2, m1b2, w1cat2):
    n = y0.shape[0]
    grid = n // BLK
    yspec = pl.BlockSpec((BLK, 32), lambda i: (i, 0))
    full = lambda s: pl.BlockSpec(s, lambda i: tuple(0 for _ in s))
    return pl.pallas_call(
        _mid_kernel,
        grid=(grid,),
        in_specs=[yspec] * 9 + [
            full((3, 32)), full((3, 32, 32)), full((3, 32)),
            full((96, 32)), full((1, 32)), full((32, 32)), full((1, 32)),
            full((32, 96)),
        ],
        out_specs=[yspec] * 3,
        out_shape=[jax.ShapeDtypeStruct((n, 32), jnp.float32)] * 3,
    )(y0, y1, y2, *ps, b1s, w2s, b2s, m1w1, m1b1, m1w2, m1b2, w1cat2)


def _fin_kernel(z0, z1, z2, q00, q01, q10, q11, q20, q21, batch_ref,
                b1s, w2s, b2s, m2w1, m2b1, m2w2, m2b2, linw, linb,
                out_ref, acc):
    i = pl.program_id(0)
    nblk = pl.num_programs(0)
    xs = []
    for k, (z, qa, qb) in enumerate(((z0, q00, q01), (z1, q10, q11),
                                     (z2, q20, q21))):
        pre = z[...] + qa[...] + qb[...] + b1s[pl.ds(k, 1)]
        t = jnp.dot(jnp.maximum(pre, 0.0), w2s[k],
                    preferred_element_type=jnp.float32) + b2s[pl.ds(k, 1)]
        xs.append(jnp.maximum(t, 0.0))
    cat = jnp.concatenate(xs, axis=1)
    v = jnp.maximum(jnp.dot(cat, m2w1[...],
                            preferred_element_type=jnp.float32) + m2b1[...], 0.0)
    v = jnp.dot(v, m2w2[...], preferred_element_type=jnp.float32) + m2b2[...]
    bb = batch_ref[0]
    g = acc.shape[0]
    oh_t = jnp.where(
        jax.lax.broadcasted_iota(jnp.int32, (g, v.shape[0]), 0) == bb,
        1.0, 0.0)
    part = jnp.dot(oh_t, v, preferred_element_type=jnp.float32)

    @pl.when(i == 0)
    def _():
        acc[...] = jnp.zeros_like(acc)

    acc[...] += part

    @pl.when(i == nblk - 1)
    def _():
        out_ref[...] = jnp.dot(acc[...], linw[...],
                               preferred_element_type=jnp.float32) + linb[...]


def _final(z0, z1, z2, qs, batch3d, b1s, w2s, b2s,
           m2w1, m2b1, m2w2, m2b2, linw, linb, g):
    n = z0.shape[0]
    grid = n // BLK
    zspec = pl.BlockSpec((BLK, 32), lambda i: (i, 0))
    full = lambda s: pl.BlockSpec(s, lambda i: tuple(0 for _ in s))
    return pl.pallas_call(
        _fin_kernel,
        grid=(grid,),
        in_specs=[zspec] * 9 + [
            pl.BlockSpec((1, 1, BLK), lambda i: (i, 0, 0)),
            full((3, 32)), full((3, 32, 32)), full((3, 32)),
            full((96, 32)), full((1, 32)), full((32, 32)), full((1, 32)),
            full((32, 1)), full((1, 1)),
        ],
        out_specs=full((g, 1)),
        out_shape=jax.ShapeDtypeStruct((g, 1), jnp.float32),
        scratch_shapes=[pltpu.VMEM((g, 32), jnp.float32)],
    )(z0, z1, z2, *qs, batch3d, b1s, w2s, b2s,
      m2w1, m2b1, m2w2, m2b2, linw, linb)


def kernel(x, edge_index_0, edge_index_1, edge_index_2, batch,
           c11_W1, c11_b1, c11_W2, c11_b2,
           c12_W1, c12_b1, c12_W2, c12_b2,
           c13_W1, c13_b1, c13_W2, c13_b2,
           c21_W1, c21_b1, c21_W2, c21_b2,
           c22_W1, c22_b1, c22_W2, c22_b2,
           c23_W1, c23_b1, c23_W2, c23_b2,
           mlp1_W1, mlp1_b1, mlp1_W2, mlp1_b2,
           mlp2_W1, mlp2_b1, mlp2_W2, mlp2_b2,
           lin_W, lin_b):
    n = x.shape[0]
    g = 64

    w1cat = jnp.concatenate([c11_W1, c12_W1, c13_W1], axis=1)
    w1cat2 = jnp.concatenate([c21_W1, c22_W1, c23_W1], axis=1)
    b1s_1 = jnp.stack([c11_b1, c12_b1, c13_b1])
    w2s_1 = jnp.stack([c11_W2, c12_W2, c13_W2])
    b2s_1 = jnp.stack([c11_b2, c12_b2, c13_b2])
    b1s_2 = jnp.stack([c21_b1, c22_b1, c23_b1])
    w2s_2 = jnp.stack([c21_W2, c22_W2, c23_W2])
    b2s_2 = jnp.stack([c21_b2, c22_b2, c23_b2])

    y0, y1, y2 = _premul3(x, w1cat)
    ps = _segsum3(y0, y1, y2, edge_index_0, edge_index_1, edge_index_2)
    z0, z1, z2 = _mid(y0, y1, y2, ps, b1s_1, w2s_1, b2s_1,
                      mlp1_W1, mlp1_b1.reshape(1, 32), mlp1_W2,
                      mlp1_b2.reshape(1, 32), w1cat2)

    qs = _segsum3(z0, z1, z2, edge_index_0, edge_index_1, edge_index_2)
    batch3d = batch.reshape(n // BLK, 1, BLK)
    out = _final(z0, z1, z2, qs, batch3d, b1s_2, w2s_2, b2s_2,
                 mlp2_W1, mlp2_b1.reshape(1, 32), mlp2_W2,
                 mlp2_b2.reshape(1, 32), lin_W, lin_b.reshape(1, 1), g)
    return jnp.squeeze(out, axis=-1)


# speedup vs baseline: 22.2931x; 1.3114x over previous
import functools

import jax
import jax.numpy as jnp
from jax import lax
from jax.experimental import pallas as pl
from jax.experimental.pallas import tpu as pltpu
from jax.experimental.pallas import tpu_sc as plsc

BLK = 2000


def _bd4(w):
    return jnp.kron(jnp.eye(4, dtype=jnp.float32), w)


def _t4(b):
    return jnp.tile(b, 4)


def _mm_kernel(x_ref, w0, w1, w2, o0, o1, o2):
    xb = x_ref[...]
    for w, o in ((w0, o0), (w1, o1), (w2, o2)):
        o[...] = jnp.dot(xb, w[...], preferred_element_type=jnp.float32)


def _premul3(x4, bdw1s):
    n4, d4 = x4.shape
    blk4 = n4
    grid = 1
    outs = [jax.ShapeDtypeStruct((n4, 128), jnp.float32)] * 3
    return pl.pallas_call(
        _mm_kernel,
        grid=(grid,),
        in_specs=[pl.BlockSpec((blk4, d4), lambda i: (i, 0))] + [
            pl.BlockSpec((d4, 128), lambda i: (0, 0))] * 3,
        out_specs=[pl.BlockSpec((blk4, 128), lambda i: (i, 0))] * 3,
        out_shape=outs,
    )(x4, *bdw1s)


def _segsum3(y0, y1, y2, e0, e1, e2):
    n = y0.shape[0]
    e = e0.shape[1]
    info = plsc.get_sparse_core_info()
    nc, ns = info.num_cores, info.num_subcores
    nw = nc * ns
    epw = e // nw
    ch = 80
    nch = epw // ch
    nb = 8
    assert ch * nch == epw and epw * nw == e
    rpt = (-(-n // ns) + 7) // 8 * 8
    n_pad = rpt * ns

    @functools.partial(
        pl.kernel,
        out_type=[jax.ShapeDtypeStruct((n_pad, 32), jnp.float32)] * 6,
        mesh=plsc.VectorSubcoreMesh(core_axis_name="c", subcore_axis_name="s"),
        scratch_types=[
            pltpu.VMEM_SHARED((n_pad, 32), jnp.float32),
            pltpu.VMEM_SHARED((n_pad, 32), jnp.float32),
            pltpu.VMEM_SHARED((n_pad, 32), jnp.float32),
            [pltpu.VMEM((ch,), jnp.int32) for _ in range(nb)],
            [pltpu.VMEM((ch,), jnp.int32) for _ in range(nb)],
            [pltpu.VMEM((ch, 32), jnp.float32) for _ in range(nb)],
            [pltpu.SemaphoreType.DMA for _ in range(nb)],
            [pltpu.SemaphoreType.DMA for _ in range(nb)],
            [pltpu.SemaphoreType.DMA for _ in range(nb)],
        ],
        compiler_params=pltpu.CompilerParams(use_tc_tiling_on_sc=False),
    )
    def k(y0h, y1h, y2h, e0h, e1h, e2h,
          o00, o01, o10, o11, o20, o21,
          a0, a1, a2, sidx, didx, rows, isem, gsem, ssem):
        cid = lax.axis_index("c")
        sid = lax.axis_index("s")
        wid = sid * nc + cid
        base = wid * epw
        rows_a = rows[0]

        def prefetch(eh, c, b):
            pltpu.async_copy(eh.at[0, pl.ds(base + c * ch, ch)],
                             sidx[b], isem[b])
            pltpu.async_copy(eh.at[1, pl.ds(base + c * ch, ch)],
                             didx[b], isem[b])

        def wait_prefetch(eh, c, b):
            pltpu.make_async_copy(eh.at[0, pl.ds(base + c * ch, ch)],
                                  sidx[b], isem[b]).wait()
            pltpu.make_async_copy(eh.at[1, pl.ds(base + c * ch, ch)],
                                  didx[b], isem[b]).wait()

        for b in range(nb):
            prefetch(e0h, b, b)

        zero16 = jnp.zeros((16,), jnp.float32)

        def zb(i, carry):
            rows_a[i, pl.ds(0, 16)] = zero16
            rows_a[i, pl.ds(16, 16)] = zero16
            return carry

        lax.fori_loop(0, ch, zb, 0)
        r0 = sid * rpt
        nfull, rem = divmod(rpt, ch)
        for a in (a0, a1, a2):
            for j in range(nfull):
                pltpu.sync_copy(rows_a, a.at[pl.ds(r0 + j * ch, ch)])
            if rem:
                pltpu.sync_copy(rows_a.at[pl.ds(0, rem)],
                                a.at[pl.ds(r0 + nfull * ch, rem)])
        plsc.subcore_barrier()

        ngrp = nch // nb
        npeel = nch - ngrp * nb
        sets = ((y0h, e0h, a0), (y1h, e1h, a1), (y2h, e2h, a2))
        for ksi, (yh, eh, a) in enumerate(sets):
            if ksi > 0:
                for b in range(nb):
                    prefetch(eh, b, b)

            def grp(g, carry):
                for b in range(nb):
                    c = g * nb + b
                    wait_prefetch(eh, c, b)
                    pltpu.async_copy(yh.at[sidx[b]], rows[b], gsem[b])
                for b in range(nb):
                    pltpu.make_async_copy(yh.at[sidx[b]], rows[b],
                                          gsem[b]).wait()
                    pltpu.async_copy(rows[b], a.at[didx[b]], ssem[b],
                                     add=True)
                for b in range(nb):
                    c_next = g * nb + b + nb

                    @pl.when(c_next < nch)
                    def _():
                        pltpu.make_async_copy(rows[b], a.at[didx[b]],
                                              ssem[b]).wait()
                        prefetch(eh, c_next, b)
                return carry

            lax.fori_loop(0, ngrp, grp, 0)
            for b in range(npeel):
                c = ngrp * nb + b
                wait_prefetch(eh, c, b)
                pltpu.async_copy(yh.at[sidx[b]], rows[b], gsem[b])
            for b in range(npeel):
                pltpu.make_async_copy(yh.at[sidx[b]], rows[b], gsem[b]).wait()
                pltpu.async_copy(rows[b], a.at[didx[b]], ssem[b], add=True)
            for b in range(nb):
                pltpu.make_async_copy(rows[b], a.at[didx[b]], ssem[b]).wait()
        plsc.subcore_barrier()

        for a, oc0, oc1 in ((a0, o00, o01), (a1, o10, o11), (a2, o20, o21)):
            @pl.when(cid == 0)
            def _():
                pltpu.sync_copy(a.at[pl.ds(r0, rpt)], oc0.at[pl.ds(r0, rpt)])

            @pl.when(cid == 1)
            def _():
                pltpu.sync_copy(a.at[pl.ds(r0, rpt)], oc1.at[pl.ds(r0, rpt)])

    return k(y0, y1, y2, e0, e1, e2)


def _mid_kernel(y0, y1, y2, p00, p01, p10, p11, p20, p21, b1s, w2s, b2s,
                m1w1s, m1b1, m1w2, m1b2, w1s2, o0, o1, o2):
    upre = m1b1[...]
    for i, (y, pa, pb) in enumerate(((y0, p00, p01), (y1, p10, p11),
                                     (y2, p20, p21))):
        nr = y.shape[0]
        pre = (y[...] + pa[pl.ds(0, nr), :] + pb[pl.ds(0, nr), :]
               + b1s[pl.ds(i, 1)])
        t = jnp.dot(jnp.maximum(pre, 0.0), w2s[i],
                    preferred_element_type=jnp.float32) + b2s[pl.ds(i, 1)]
        upre = upre + jnp.dot(t, m1w1s[i], preferred_element_type=jnp.float32)
    u = jnp.dot(jnp.maximum(upre, 0.0), m1w2[...],
                preferred_element_type=jnp.float32) + m1b2[...]
    for i, o in enumerate((o0, o1, o2)):
        o[...] = jnp.dot(u, w1s2[i], preferred_element_type=jnp.float32)


def _mid(y0, y1, y2, ps, b1s, w2s, b2s, m1w1s, m1b1, m1w2, m1b2, w1s2):
    n4 = y0.shape[0]
    blk4 = n4
    grid = 1
    yspec = pl.BlockSpec((blk4, 128), lambda i: (i, 0))
    pspec = pl.BlockSpec(ps[0].shape, lambda i: (0, 0))
    full = lambda s: pl.BlockSpec(s, lambda i: tuple(0 for _ in s))
    return pl.pallas_call(
        _mid_kernel,
        grid=(grid,),
        in_specs=[yspec] * 3 + [pspec] * 6 + [
            full((3, 128)), full((3, 128, 128)), full((3, 128)),
            full((3, 128, 128)), full((1, 128)), full((128, 128)),
            full((1, 128)), full((3, 128, 128)),
        ],
        out_specs=[yspec] * 3,
        out_shape=[jax.ShapeDtypeStruct((n4, 128), jnp.float32)] * 3,
    )(y0, y1, y2, *ps, b1s, w2s, b2s, m1w1s, m1b1, m1w2, m1b2, w1s2)


def _fin_kernel(z0, z1, z2, q00, q01, q10, q11, q20, q21, batch_ref,
                b1s, w2s, b2s, m2w1s, m2b1, m2w2, m2b2, linw, linb,
                out_ref, acc):
    i = pl.program_id(0)
    nblk = pl.num_programs(0)
    vpre = m2b1[...]
    for k, (z, qa, qb) in enumerate(((z0, q00, q01), (z1, q10, q11),
                                     (z2, q20, q21))):
        nr = z.shape[0]
        pre = (z[...] + qa[pl.ds(0, nr), :] + qb[pl.ds(0, nr), :]
               + b1s[pl.ds(k, 1)])
        t = jnp.dot(jnp.maximum(pre, 0.0), w2s[k],
                    preferred_element_type=jnp.float32) + b2s[pl.ds(k, 1)]
        vpre = vpre + jnp.dot(jnp.maximum(t, 0.0), m2w1s[k],
                              preferred_element_type=jnp.float32)
    v = jnp.dot(jnp.maximum(vpre, 0.0), m2w2[...],
                preferred_element_type=jnp.float32) + m2b2[...]
    g = acc.shape[0]
    blk4 = v.shape[0]
    part = jnp.zeros((g, 32), jnp.float32)
    for j in range(4):
        bj = batch_ref[:, pl.ds(j, 1)]
        ohjt = jnp.where(
            jax.lax.broadcasted_iota(jnp.int32, (blk4, g), 1) == bj, 1.0, 0.0)
        vj = v[:, 32 * j:32 * j + 32]
        part = part + jax.lax.dot_general(
            ohjt, vj, (((0,), (0,)), ((), ())),
            preferred_element_type=jnp.float32)

    @pl.when(i == 0)
    def _():
        acc[...] = jnp.zeros_like(acc)

    acc[...] += part

    @pl.when(i == nblk - 1)
    def _():
        out_ref[...] = jnp.dot(acc[...], linw[...],
                               preferred_element_type=jnp.float32) + linb[...]


def _final(z0, z1, z2, qs, batch4, b1s, w2s, b2s,
           m2w1s, m2b1, m2w2, m2b2, linw, linb, g):
    n4 = z0.shape[0]
    blk4 = n4
    grid = 1
    zspec = pl.BlockSpec((blk4, 128), lambda i: (i, 0))
    qspec = pl.BlockSpec(qs[0].shape, lambda i: (0, 0))
    full = lambda s: pl.BlockSpec(s, lambda i: tuple(0 for _ in s))
    return pl.pallas_call(
        _fin_kernel,
        grid=(grid,),
        in_specs=[zspec] * 3 + [qspec] * 6 + [
            pl.BlockSpec((blk4, 4), lambda i: (i, 0)),
            full((3, 128)), full((3, 128, 128)), full((3, 128)),
            full((3, 128, 128)), full((1, 128)), full((128, 128)),
            full((1, 128)), full((32, 1)), full((1, 1)),
        ],
        out_specs=full((g, 1)),
        out_shape=jax.ShapeDtypeStruct((g, 1), jnp.float32),
        scratch_shapes=[pltpu.VMEM((g, 32), jnp.float32)],
    )(z0, z1, z2, *qs, batch4, b1s, w2s, b2s,
      m2w1s, m2b1, m2w2, m2b2, linw, linb)


def kernel(x, edge_index_0, edge_index_1, edge_index_2, batch,
           c11_W1, c11_b1, c11_W2, c11_b2,
           c12_W1, c12_b1, c12_W2, c12_b2,
           c13_W1, c13_b1, c13_W2, c13_b2,
           c21_W1, c21_b1, c21_W2, c21_b2,
           c22_W1, c22_b1, c22_W2, c22_b2,
           c23_W1, c23_b1, c23_W2, c23_b2,
           mlp1_W1, mlp1_b1, mlp1_W2, mlp1_b2,
           mlp2_W1, mlp2_b1, mlp2_W2, mlp2_b2,
           lin_W, lin_b):
    n = x.shape[0]
    g = 64

    bdw1s = [jnp.kron(jnp.eye(4, dtype=jnp.float32), w)
             for w in (c11_W1, c12_W1, c13_W1)]
    b1s_1 = jnp.stack([_t4(c11_b1), _t4(c12_b1), _t4(c13_b1)])
    w2s_1 = jnp.stack([_bd4(c11_W2), _bd4(c12_W2), _bd4(c13_W2)])
    b2s_1 = jnp.stack([_t4(c11_b2), _t4(c12_b2), _t4(c13_b2)])
    b1s_2 = jnp.stack([_t4(c21_b1), _t4(c22_b1), _t4(c23_b1)])
    w2s_2 = jnp.stack([_bd4(c21_W2), _bd4(c22_W2), _bd4(c23_W2)])
    b2s_2 = jnp.stack([_t4(c21_b2), _t4(c22_b2), _t4(c23_b2)])
    m1w1s = jnp.stack([_bd4(mlp1_W1[32 * k:32 * k + 32]) for k in range(3)])
    m2w1s = jnp.stack([_bd4(mlp2_W1[32 * k:32 * k + 32]) for k in range(3)])
    m1w2bd = _bd4(mlp1_W2)
    m2w2bd = _bd4(mlp2_W2)
    w1s2 = jnp.stack([_bd4(c21_W1), _bd4(c22_W1), _bd4(c23_W1)])

    x4 = x.reshape(n // 4, 512)
    y0p, y1p, y2p = _premul3(x4, bdw1s)
    ps = _segsum3(y0p.reshape(n, 32), y1p.reshape(n, 32), y2p.reshape(n, 32),
                  edge_index_0, edge_index_1, edge_index_2)
    ps4 = [p.reshape(p.shape[0] // 4, 128) for p in ps]
    z0p, z1p, z2p = _mid(y0p, y1p, y2p, ps4, b1s_1, w2s_1, b2s_1,
                         m1w1s, _t4(mlp1_b1).reshape(1, 128), m1w2bd,
                         _t4(mlp1_b2).reshape(1, 128), w1s2)

    qs = _segsum3(z0p.reshape(n, 32), z1p.reshape(n, 32), z2p.reshape(n, 32),
                  edge_index_0, edge_index_1, edge_index_2)
    qs4 = [q.reshape(q.shape[0] // 4, 128) for q in qs]
    batch4 = batch.reshape(n // 4, 4)
    out = _final(z0p, z1p, z2p, qs4, batch4, b1s_2, w2s_2, b2s_2,
                 m2w1s, _t4(mlp2_b1).reshape(1, 128), m2w2bd,
                 _t4(mlp2_b2).reshape(1, 128), lin_W, lin_b.reshape(1, 1), g)
    return jnp.squeeze(out, axis=-1)


# speedup vs baseline: 24.3004x; 1.0900x over previous
import functools

import jax
import jax.numpy as jnp
from jax import lax
from jax.experimental import pallas as pl
from jax.experimental.pallas import tpu as pltpu
from jax.experimental.pallas import tpu_sc as plsc

BLK = 2000


def _bd4(w):
    return jnp.kron(jnp.eye(4, dtype=jnp.float32), w)


def _t4(b):
    return jnp.tile(b, 4)


def _mm_kernel(x_ref, w0, w1, w2, o0, o1, o2):
    xb = x_ref[...]
    for w, o in ((w0, o0), (w1, o1), (w2, o2)):
        o[...] = jnp.dot(xb, w[...], preferred_element_type=jnp.float32)


def _premul3(x4, bdw1s):
    n4, d4 = x4.shape
    blk4 = n4
    grid = 1
    outs = [jax.ShapeDtypeStruct((n4, 128), jnp.float32)] * 3
    return pl.pallas_call(
        _mm_kernel,
        grid=(grid,),
        in_specs=[pl.BlockSpec((blk4, d4), lambda i: (i, 0))] + [
            pl.BlockSpec((d4, 128), lambda i: (0, 0))] * 3,
        out_specs=[pl.BlockSpec((blk4, 128), lambda i: (i, 0))] * 3,
        out_shape=outs,
    )(x4, *bdw1s)


def _segsum3(y0, y1, y2, e0, e1, e2):
    n = y0.shape[0]
    e = e0.shape[1]
    info = plsc.get_sparse_core_info()
    nc, ns = info.num_cores, info.num_subcores
    nw = nc * ns
    epw = e // nw
    ch = 200
    nch = epw // ch
    nb = 8
    assert ch * nch == epw and epw * nw == e
    rpt = (-(-n // ns) + 7) // 8 * 8
    n_pad = rpt * ns

    @functools.partial(
        pl.kernel,
        out_type=[jax.ShapeDtypeStruct((n_pad, 32), jnp.float32)] * 6,
        mesh=plsc.VectorSubcoreMesh(core_axis_name="c", subcore_axis_name="s"),
        scratch_types=[
            pltpu.VMEM_SHARED((n_pad, 32), jnp.float32),
            pltpu.VMEM_SHARED((n_pad, 32), jnp.float32),
            pltpu.VMEM_SHARED((n_pad, 32), jnp.float32),
            [pltpu.VMEM((ch,), jnp.int32) for _ in range(nb)],
            [pltpu.VMEM((ch,), jnp.int32) for _ in range(nb)],
            [pltpu.VMEM((ch, 32), jnp.float32) for _ in range(nb)],
            [pltpu.SemaphoreType.DMA for _ in range(nb)],
            [pltpu.SemaphoreType.DMA for _ in range(nb)],
            [pltpu.SemaphoreType.DMA for _ in range(nb)],
        ],
        compiler_params=pltpu.CompilerParams(use_tc_tiling_on_sc=False),
    )
    def k(y0h, y1h, y2h, e0h, e1h, e2h,
          o00, o01, o10, o11, o20, o21,
          a0, a1, a2, sidx, didx, rows, isem, gsem, ssem):
        cid = lax.axis_index("c")
        sid = lax.axis_index("s")
        wid = sid * nc + cid
        base = wid * epw
        rows_a = rows[0]

        def prefetch(eh, c, b):
            pltpu.async_copy(eh.at[0, pl.ds(base + c * ch, ch)],
                             sidx[b], isem[b])
            pltpu.async_copy(eh.at[1, pl.ds(base + c * ch, ch)],
                             didx[b], isem[b])

        def wait_prefetch(eh, c, b):
            pltpu.make_async_copy(eh.at[0, pl.ds(base + c * ch, ch)],
                                  sidx[b], isem[b]).wait()
            pltpu.make_async_copy(eh.at[1, pl.ds(base + c * ch, ch)],
                                  didx[b], isem[b]).wait()

        for b in range(nb):
            prefetch(e0h, b, b)

        zero16 = jnp.zeros((16,), jnp.float32)

        def zb(i, carry):
            rows_a[i, pl.ds(0, 16)] = zero16
            rows_a[i, pl.ds(16, 16)] = zero16
            return carry

        lax.fori_loop(0, ch, zb, 0)
        r0 = sid * rpt
        nfull, rem = divmod(rpt, ch)
        zdscs = []
        zi = 0
        for a in (a0, a1, a2):
            for j in range(nfull):
                zdscs.append(pltpu.async_copy(
                    rows_a, a.at[pl.ds(r0 + j * ch, ch)], gsem[zi % nb]))
                zi += 1
            if rem:
                zdscs.append(pltpu.async_copy(
                    rows_a.at[pl.ds(0, rem)],
                    a.at[pl.ds(r0 + nfull * ch, rem)], gsem[zi % nb]))
                zi += 1
        for dsc in zdscs:
            dsc.wait()
        plsc.subcore_barrier()

        ngrp = nch // nb
        npeel = nch - ngrp * nb
        sets = ((y0h, e0h, a0), (y1h, e1h, a1), (y2h, e2h, a2))
        for ksi, (yh, eh, a) in enumerate(sets):
            if ksi > 0:
                for b in range(nb):
                    prefetch(eh, b, b)

            def grp(g, carry):
                for b in range(nb):
                    c = g * nb + b
                    wait_prefetch(eh, c, b)
                    pltpu.async_copy(yh.at[sidx[b]], rows[b], gsem[b])
                for b in range(nb):
                    pltpu.make_async_copy(yh.at[sidx[b]], rows[b],
                                          gsem[b]).wait()
                    pltpu.async_copy(rows[b], a.at[didx[b]], ssem[b],
                                     add=True)
                for b in range(nb):
                    c_next = g * nb + b + nb

                    @pl.when(c_next < nch)
                    def _():
                        pltpu.make_async_copy(rows[b], a.at[didx[b]],
                                              ssem[b]).wait()
                        prefetch(eh, c_next, b)
                return carry

            lax.fori_loop(0, ngrp, grp, 0)
            for b in range(npeel):
                c = ngrp * nb + b
                wait_prefetch(eh, c, b)
                pltpu.async_copy(yh.at[sidx[b]], rows[b], gsem[b])
            for b in range(npeel):
                pltpu.make_async_copy(yh.at[sidx[b]], rows[b], gsem[b]).wait()
                pltpu.async_copy(rows[b], a.at[didx[b]], ssem[b], add=True)
            for b in range(nb):
                pltpu.make_async_copy(rows[b], a.at[didx[b]], ssem[b]).wait()
        plsc.subcore_barrier()

        for a, oc0, oc1 in ((a0, o00, o01), (a1, o10, o11), (a2, o20, o21)):
            @pl.when(cid == 0)
            def _():
                pltpu.sync_copy(a.at[pl.ds(r0, rpt)], oc0.at[pl.ds(r0, rpt)])

            @pl.when(cid == 1)
            def _():
                pltpu.sync_copy(a.at[pl.ds(r0, rpt)], oc1.at[pl.ds(r0, rpt)])

    return k(y0, y1, y2, e0, e1, e2)


def _mid_kernel(y0, y1, y2, p00, p01, p10, p11, p20, p21, b1s, w2s, b2s,
                m1w1s, m1b1, m1w2, m1b2, w1s2, o0, o1, o2):
    upre = m1b1[...]
    for i, (y, pa, pb) in enumerate(((y0, p00, p01), (y1, p10, p11),
                                     (y2, p20, p21))):
        nr = y.shape[0]
        pre = (y[...] + pa[pl.ds(0, nr), :] + pb[pl.ds(0, nr), :]
               + b1s[pl.ds(i, 1)])
        t = jnp.dot(jnp.maximum(pre, 0.0), w2s[i],
                    preferred_element_type=jnp.float32) + b2s[pl.ds(i, 1)]
        upre = upre + jnp.dot(t, m1w1s[i], preferred_element_type=jnp.float32)
    u = jnp.dot(jnp.maximum(upre, 0.0), m1w2[...],
                preferred_element_type=jnp.float32) + m1b2[...]
    for i, o in enumerate((o0, o1, o2)):
        o[...] = jnp.dot(u, w1s2[i], preferred_element_type=jnp.float32)


def _mid(y0, y1, y2, ps, b1s, w2s, b2s, m1w1s, m1b1, m1w2, m1b2, w1s2):
    n4 = y0.shape[0]
    blk4 = n4
    grid = 1
    yspec = pl.BlockSpec((blk4, 128), lambda i: (i, 0))
    pspec = pl.BlockSpec(ps[0].shape, lambda i: (0, 0))
    full = lambda s: pl.BlockSpec(s, lambda i: tuple(0 for _ in s))
    return pl.pallas_call(
        _mid_kernel,
        grid=(grid,),
        in_specs=[yspec] * 3 + [pspec] * 6 + [
            full((3, 128)), full((3, 128, 128)), full((3, 128)),
            full((3, 128, 128)), full((1, 128)), full((128, 128)),
            full((1, 128)), full((3, 128, 128)),
        ],
        out_specs=[yspec] * 3,
        out_shape=[jax.ShapeDtypeStruct((n4, 128), jnp.float32)] * 3,
    )(y0, y1, y2, *ps, b1s, w2s, b2s, m1w1s, m1b1, m1w2, m1b2, w1s2)


def _fin_kernel(z0, z1, z2, q00, q01, q10, q11, q20, q21, batch_ref,
                b1s, w2s, b2s, m2w1s, m2b1, m2w2, m2b2, linw, linb,
                out_ref, acc):
    i = pl.program_id(0)
    nblk = pl.num_programs(0)
    vpre = m2b1[...]
    for k, (z, qa, qb) in enumerate(((z0, q00, q01), (z1, q10, q11),
                                     (z2, q20, q21))):
        nr = z.shape[0]
        pre = (z[...] + qa[pl.ds(0, nr), :] + qb[pl.ds(0, nr), :]
               + b1s[pl.ds(k, 1)])
        t = jnp.dot(jnp.maximum(pre, 0.0), w2s[k],
                    preferred_element_type=jnp.float32) + b2s[pl.ds(k, 1)]
        vpre = vpre + jnp.dot(jnp.maximum(t, 0.0), m2w1s[k],
                              preferred_element_type=jnp.float32)
    v = jnp.dot(jnp.maximum(vpre, 0.0), m2w2[...],
                preferred_element_type=jnp.float32) + m2b2[...]
    g = acc.shape[0]
    blk4 = v.shape[0]
    part = jnp.zeros((g, 32), jnp.float32)
    for j in range(4):
        bj = batch_ref[:, pl.ds(j, 1)]
        ohjt = jnp.where(
            jax.lax.broadcasted_iota(jnp.int32, (blk4, g), 1) == bj, 1.0, 0.0)
        vj = v[:, 32 * j:32 * j + 32]
        part = part + jax.lax.dot_general(
            ohjt, vj, (((0,), (0,)), ((), ())),
            preferred_element_type=jnp.float32)

    @pl.when(i == 0)
    def _():
        acc[...] = jnp.zeros_like(acc)

    acc[...] += part

    @pl.when(i == nblk - 1)
    def _():
        out_ref[...] = jnp.dot(acc[...], linw[...],
                               preferred_element_type=jnp.float32) + linb[...]


def _final(z0, z1, z2, qs, batch4, b1s, w2s, b2s,
           m2w1s, m2b1, m2w2, m2b2, linw, linb, g):
    n4 = z0.shape[0]
    blk4 = n4
    grid = 1
    zspec = pl.BlockSpec((blk4, 128), lambda i: (i, 0))
    qspec = pl.BlockSpec(qs[0].shape, lambda i: (0, 0))
    full = lambda s: pl.BlockSpec(s, lambda i: tuple(0 for _ in s))
    return pl.pallas_call(
        _fin_kernel,
        grid=(grid,),
        in_specs=[zspec] * 3 + [qspec] * 6 + [
            pl.BlockSpec((blk4, 4), lambda i: (i, 0)),
            full((3, 128)), full((3, 128, 128)), full((3, 128)),
            full((3, 128, 128)), full((1, 128)), full((128, 128)),
            full((1, 128)), full((32, 1)), full((1, 1)),
        ],
        out_specs=full((g, 1)),
        out_shape=jax.ShapeDtypeStruct((g, 1), jnp.float32),
        scratch_shapes=[pltpu.VMEM((g, 32), jnp.float32)],
    )(z0, z1, z2, *qs, batch4, b1s, w2s, b2s,
      m2w1s, m2b1, m2w2, m2b2, linw, linb)


def kernel(x, edge_index_0, edge_index_1, edge_index_2, batch,
           c11_W1, c11_b1, c11_W2, c11_b2,
           c12_W1, c12_b1, c12_W2, c12_b2,
           c13_W1, c13_b1, c13_W2, c13_b2,
           c21_W1, c21_b1, c21_W2, c21_b2,
           c22_W1, c22_b1, c22_W2, c22_b2,
           c23_W1, c23_b1, c23_W2, c23_b2,
           mlp1_W1, mlp1_b1, mlp1_W2, mlp1_b2,
           mlp2_W1, mlp2_b1, mlp2_W2, mlp2_b2,
           lin_W, lin_b):
    n = x.shape[0]
    g = 64

    bdw1s = [jnp.kron(jnp.eye(4, dtype=jnp.float32), w)
             for w in (c11_W1, c12_W1, c13_W1)]
    b1s_1 = jnp.stack([_t4(c11_b1), _t4(c12_b1), _t4(c13_b1)])
    w2s_1 = jnp.stack([_bd4(c11_W2), _bd4(c12_W2), _bd4(c13_W2)])
    b2s_1 = jnp.stack([_t4(c11_b2), _t4(c12_b2), _t4(c13_b2)])
    b1s_2 = jnp.stack([_t4(c21_b1), _t4(c22_b1), _t4(c23_b1)])
    w2s_2 = jnp.stack([_bd4(c21_W2), _bd4(c22_W2), _bd4(c23_W2)])
    b2s_2 = jnp.stack([_t4(c21_b2), _t4(c22_b2), _t4(c23_b2)])
    m1w1s = jnp.stack([_bd4(mlp1_W1[32 * k:32 * k + 32]) for k in range(3)])
    m2w1s = jnp.stack([_bd4(mlp2_W1[32 * k:32 * k + 32]) for k in range(3)])
    m1w2bd = _bd4(mlp1_W2)
    m2w2bd = _bd4(mlp2_W2)
    w1s2 = jnp.stack([_bd4(c21_W1), _bd4(c22_W1), _bd4(c23_W1)])

    x4 = x.reshape(n // 4, 512)
    y0p, y1p, y2p = _premul3(x4, bdw1s)
    ps = _segsum3(y0p.reshape(n, 32), y1p.reshape(n, 32), y2p.reshape(n, 32),
                  edge_index_0, edge_index_1, edge_index_2)
    ps4 = [p.reshape(p.shape[0] // 4, 128) for p in ps]
    z0p, z1p, z2p = _mid(y0p, y1p, y2p, ps4, b1s_1, w2s_1, b2s_1,
                         m1w1s, _t4(mlp1_b1).reshape(1, 128), m1w2bd,
                         _t4(mlp1_b2).reshape(1, 128), w1s2)

    qs = _segsum3(z0p.reshape(n, 32), z1p.reshape(n, 32), z2p.reshape(n, 32),
                  edge_index_0, edge_index_1, edge_index_2)
    qs4 = [q.reshape(q.shape[0] // 4, 128) for q in qs]
    batch4 = batch.reshape(n // 4, 4)
    out = _final(z0p, z1p, z2p, qs4, batch4, b1s_2, w2s_2, b2s_2,
                 m2w1s, _t4(mlp2_b1).reshape(1, 128), m2w2bd,
                 _t4(mlp2_b2).reshape(1, 128), lin_W, lin_b.reshape(1, 1), g)
    return jnp.squeeze(out, axis=-1)
